# Initial kernel scaffold; baseline (speedup 1.0000x reference)
import itertools

import jax
import jax.numpy as jnp
import numpy as np
from jax.experimental import pallas as pl

_DIM = 4096
_BLOCK_SIZE = 10
_HAMMING_DIST = 2
_BATCH = 64
_HIDDEN = 64
_KP = 128
_NB = 64


def _hamming_ball_np(n, k):
    ball = [np.zeros((n,))]
    for i in range(1, k + 1):
        for tup in itertools.combinations(range(n), i):
            vec = np.zeros((n,))
            vec[list(tup)] = 1.0
            ball.append(vec)
    return np.stack(ball).astype(np.float32)


def _constants():
    h = _hamming_ball_np(_BLOCK_SIZE, min(_HAMMING_DIST, _BLOCK_SIZE))
    n_ball = h.shape[0]
    h_pad = np.zeros((_NB, _KP), np.float32)
    h_pad[:n_ball, :_BLOCK_SIZE] = h

    key = jax.random.key(42)
    kc, ks = jax.random.split(key)
    chosen = jax.random.randint(kc, (_BATCH,), 0, n_ball)
    changes = jnp.take(jnp.asarray(h), chosen, axis=0)
    changes_pad = jnp.zeros((_BATCH, _KP), jnp.float32)
    changes_pad = changes_pad.at[:, :_BLOCK_SIZE].set(changes)

    g = jax.random.gumbel(ks, (_BATCH, n_ball), dtype=jnp.float32)
    g_pad = jnp.full((_BATCH, _NB), -1e30, jnp.float32)
    g_pad = g_pad.at[:, :n_ball].set(g)
    return jnp.asarray(h_pad), changes_pad, g_pad


def _sampler_kernel(x_ref, w1_ref, w2_ref, h_ref, ch_ref, g_ref, out_ref):
    x = x_ref[...]
    w1 = w1_ref[...]
    w2 = w2_ref[...]
    h = h_ref[...]
    changes = ch_ref[...]
    g = g_ref[...]

    xk = x[:, :_KP]
    ub = changes * (1.0 - xk) + (1.0 - changes) * xk
    s = 1.0 - 2.0 * ub
    w1k = w1[:_KP, :]

    base = jnp.dot(x, w1, preferred_element_type=jnp.float32)
    base = base + jnp.dot(ub - xk, w1k, preferred_element_type=jnp.float32)

    hs = (h[None, :, :] * s[:, None, :]).reshape(_BATCH * _NB, _KP)
    delta = jnp.dot(hs, w1k, preferred_element_type=jnp.float32)
    delta = delta.reshape(_BATCH, _NB, _HIDDEN)

    act = jnp.maximum(base[:, None, :] + delta, 0.0)
    logits = jnp.sum(act * w2[0][None, None, :], axis=-1)

    score = logits + g
    m = jnp.max(score, axis=1, keepdims=True)
    jidx = jax.lax.broadcasted_iota(jnp.float32, (_BATCH, _NB), 1)
    first = jnp.min(jnp.where(score == m, jidx, float(_NB)), axis=1,
                    keepdims=True)
    onehot = (jidx == first).astype(jnp.float32)

    hc = jnp.dot(onehot, h, preferred_element_type=jnp.float32)
    out_block = ub + hc * s
    out_ref[:, :_KP] = out_block
    out_ref[:, _KP:] = x[:, _KP:]


def kernel(x, W1, w2):
    h_pad, changes_pad, g_pad = _constants()
    return pl.pallas_call(
        _sampler_kernel,
        out_shape=jax.ShapeDtypeStruct((_BATCH, _DIM), jnp.float32),
    )(x, W1, w2.reshape(1, _HIDDEN), h_pad, changes_pad, g_pad)


# speedup vs baseline: 3.1068x; 3.1068x over previous
import itertools

import jax
import jax.numpy as jnp
import numpy as np
from jax.experimental import pallas as pl

_DIM = 4096
_BLOCK_SIZE = 10
_HAMMING_DIST = 2
_BATCH = 64
_HIDDEN = 64
_KP = 128
_NB = 64


def _hamming_ball_np(n, k):
    ball = [np.zeros((n,))]
    for i in range(1, k + 1):
        for tup in itertools.combinations(range(n), i):
            vec = np.zeros((n,))
            vec[list(tup)] = 1.0
            ball.append(vec)
    return np.stack(ball).astype(np.float32)


def _constants():
    h = _hamming_ball_np(_BLOCK_SIZE, min(_HAMMING_DIST, _BLOCK_SIZE))
    n_ball = h.shape[0]
    h_pad = np.zeros((_NB, _KP), np.float32)
    h_pad[:n_ball, :_BLOCK_SIZE] = h

    key = jax.random.key(42)
    kc, ks = jax.random.split(key)
    chosen = jax.random.randint(kc, (_BATCH,), 0, n_ball)
    changes = jnp.take(jnp.asarray(h), chosen, axis=0)
    changes_pad = jnp.zeros((_BATCH, _KP), jnp.float32)
    changes_pad = changes_pad.at[:, :_BLOCK_SIZE].set(changes)

    g = jax.random.gumbel(ks, (_BATCH, n_ball), dtype=jnp.float32)
    g_pad = jnp.full((_BATCH, _NB), -1e30, jnp.float32)
    g_pad = g_pad.at[:, :n_ball].set(g)
    return jnp.asarray(h_pad), changes_pad, g_pad


def _sampler_kernel(x_ref, w1_ref, w2_ref, h_ref, ch_ref, g_ref, out_ref):
    x = x_ref[...]
    w1 = w1_ref[...]
    w2 = w2_ref[...]
    h = h_ref[...]
    changes = ch_ref[...]
    g = g_ref[...]

    xk = x[:, :_KP]
    ub = changes * (1.0 - xk) + (1.0 - changes) * xk
    s = 1.0 - 2.0 * ub
    w1k = w1[:_KP, :]

    base = jnp.dot(x, w1, preferred_element_type=jnp.float32)
    base = base + jnp.dot(ub - xk, w1k, preferred_element_type=jnp.float32)

    hs = (h[None, :, :] * s[:, None, :]).reshape(_BATCH * _NB, _KP)
    delta = jnp.dot(hs, w1k, preferred_element_type=jnp.float32)
    delta = delta.reshape(_BATCH, _NB, _HIDDEN)

    act = jnp.maximum(base[:, None, :] + delta, 0.0)
    logits = jnp.sum(act * w2[0][None, None, :], axis=-1)

    score = logits + g
    m = jnp.max(score, axis=1, keepdims=True)
    jidx = jax.lax.broadcasted_iota(jnp.int32, (_BATCH, _NB), 1)
    first = jnp.min(jnp.where(score == m, jidx, _NB), axis=1,
                    keepdims=True)
    onehot = (jidx == first).astype(jnp.float32)

    hc = jnp.dot(onehot, h, preferred_element_type=jnp.float32)
    out_block = ub + hc * s
    out_ref[:, :_KP] = out_block
    out_ref[:, _KP:] = x[:, _KP:]


def kernel(x, W1, w2):
    h_pad, changes_pad, g_pad = _constants()
    return pl.pallas_call(
        _sampler_kernel,
        out_shape=jax.ShapeDtypeStruct((_BATCH, _DIM), jnp.float32),
    )(x, W1, w2.reshape(1, _HIDDEN), h_pad, changes_pad, g_pad)


# speedup vs baseline: 11.1887x; 3.6014x over previous
import itertools

import jax
import jax.numpy as jnp
import numpy as np
from jax.experimental import pallas as pl

_DIM = 4096
_BLOCK_SIZE = 10
_HAMMING_DIST = 2
_BATCH = 64
_HIDDEN = 64
_KP = 128
_NB = 64


def _hamming_ball_np(n, k):
    ball = [np.zeros((n,))]
    for i in range(1, k + 1):
        for tup in itertools.combinations(range(n), i):
            vec = np.zeros((n,))
            vec[list(tup)] = 1.0
            ball.append(vec)
    return np.stack(ball).astype(np.float32)


def _constants():
    h = _hamming_ball_np(_BLOCK_SIZE, min(_HAMMING_DIST, _BLOCK_SIZE))
    n_ball = h.shape[0]
    h_pad = np.zeros((_NB, _KP), np.float32)
    h_pad[:n_ball, :_BLOCK_SIZE] = h

    try:
        dev = jax.local_devices(backend="cpu")[0]
    except RuntimeError:
        dev = None
    with jax.default_device(dev):
        key = jax.random.key(42)
        kc, ks = jax.random.split(key)
        chosen = np.asarray(jax.random.randint(kc, (_BATCH,), 0, n_ball))
        g = np.asarray(jax.random.gumbel(ks, (_BATCH, n_ball),
                                         dtype=jnp.float32))

    changes_pad = np.zeros((_BATCH, _KP), np.float32)
    changes_pad[:, :_BLOCK_SIZE] = h[chosen]

    g_pad = np.full((_BATCH, _NB), -1e30, np.float32)
    g_pad[:, :n_ball] = g
    return h_pad, changes_pad, g_pad


_H_PAD, _CHANGES_PAD, _G_PAD = _constants()


def _sampler_kernel(x_ref, w1_ref, w2_ref, h_ref, ch_ref, g_ref, out_ref):
    x = x_ref[...]
    w1 = w1_ref[...]
    w2 = w2_ref[...]
    h = h_ref[...]
    changes = ch_ref[...]
    g = g_ref[...]

    xk = x[:, :_KP]
    ub = changes * (1.0 - xk) + (1.0 - changes) * xk
    s = 1.0 - 2.0 * ub
    w1k = w1[:_KP, :]

    base = jnp.dot(x, w1, preferred_element_type=jnp.float32)
    base = base + jnp.dot(ub - xk, w1k, preferred_element_type=jnp.float32)

    hs = (h[None, :, :] * s[:, None, :]).reshape(_BATCH * _NB, _KP)
    delta = jnp.dot(hs, w1k, preferred_element_type=jnp.float32)
    delta = delta.reshape(_BATCH, _NB, _HIDDEN)

    act = jnp.maximum(base[:, None, :] + delta, 0.0)
    logits = jnp.sum(act * w2[0][None, None, :], axis=-1)

    score = logits + g
    m = jnp.max(score, axis=1, keepdims=True)
    jidx = jax.lax.broadcasted_iota(jnp.int32, (_BATCH, _NB), 1)
    first = jnp.min(jnp.where(score == m, jidx, _NB), axis=1,
                    keepdims=True)
    onehot = (jidx == first).astype(jnp.float32)

    hc = jnp.dot(onehot, h, preferred_element_type=jnp.float32)
    out_block = ub + hc * s
    out_ref[:, :_KP] = out_block
    out_ref[:, _KP:] = x[:, _KP:]


def kernel(x, W1, w2):
    return pl.pallas_call(
        _sampler_kernel,
        out_shape=jax.ShapeDtypeStruct((_BATCH, _DIM), jnp.float32),
    )(x, W1, w2.reshape(1, _HIDDEN), _H_PAD, _CHANGES_PAD, _G_PAD)


# speedup vs baseline: 17.3206x; 1.5480x over previous
import base64
import itertools

import jax
import jax.numpy as jnp
import numpy as np
from jax.experimental import pallas as pl
from jax.experimental.pallas import tpu as pltpu

_DIM = 4096
_BLOCK_SIZE = 10
_HAMMING_DIST = 2
_BATCH = 64
_HIDDEN = 64
_KP = 128
_NB = 64


def _hamming_ball_np(n, k):
    ball = [np.zeros((n,))]
    for i in range(1, k + 1):
        for tup in itertools.combinations(range(n), i):
            vec = np.zeros((n,))
            vec[list(tup)] = 1.0
            ball.append(vec)
    return np.stack(ball).astype(np.float32)


_CHOSEN = np.array([
    36, 51, 51, 33, 14, 35, 52, 53, 28, 27, 28, 53, 50, 11, 19, 0,
    3, 31, 51, 11, 37, 41, 0, 11, 23, 13, 15, 36, 20, 25, 44, 51,
    16, 47, 27, 28, 12, 17, 25, 29, 6, 8, 50, 34, 8, 33, 18, 40,
    1, 8, 23, 23, 11, 31, 19, 32, 47, 21, 40, 53, 48, 43, 32, 20,
], np.int32)

_G_B64 = (
    "Ia+SP3WDtz8+qAi/07OUvjHsSb+Zeu++yigMvt1pT7/MQS0+F+yePlHzmb3iaWm9TJsTQKzZ"
    "Hz9wjGA/zLpKPe/QFL88gW0/x03UvXXMDL8Nm4s9McSgP+MhF74SDYE/9wRTPltj7T7Ijqs/"
    "2KQMQBy+uD9dPzO/s+WpvsNhhb+vbQ0934VaP5M0Ar+o9My/PGvSP1mwD78K5XQ/rgBpvkTB"
    "pj5H15s/uHyovo0H8j177Bi/kRuxvwEkWEDvTVU/NlvIP/s2aT3iw349FncCQDw0ID/Xpk+9"
    "7snzvRRr/z+PYGi+7S+xPxgHFL8M0so+XT5LP00NDL/G+jU/M6lpv9ErnD+VkE+/WDnrPvV+"
    "eL5FZC9AF3QkvrC5j77zzqk+23fMvlq9FT8YgQ1Af2rcPpQHQz/Mzea9MDs/vwulWj+Vsxy+"
    "cKdoPrIIU7+MCiW/fAQfQH/CcL/CJXg/AGRBP7BsMT+G4rI/aGjovqDm8z9LRyY+TEnCPn1B"
    "grzSCDtAqhR3Pjgs5j5qoQdAdB2Vv1bq6z8170i/5uFAv1joAEAwvThAZPCCP/nHmD6xeAtA"
    "4Y4qv07jHkBR2M6+2C0Nv7yGg7+//7s/rT9Yv2eatz8dt6o/UQ5ov5ZyAL45MNw/BDXiPyYE"
    "479g2+k/phATvyitqj9uobY/+Vy1P6HOBUA7/5k+ydIkv2AQyr+kzOE+JJxAP4Fumj+fQhhA"
    "ylqGPhwPDkAqsHc9ZmmKPWS1ob+qT0M+5Ag0QIeQBr8dRy5A5yORvxiolD9mFbM/ZX3Zvfwh"
    "vb/fdjJAMI03P9do3D9NjY2/2nSqvnz7fz5nxsu+EQ/7vuvJPL+Gy84+tHs9PYF+jz4hqqE/"
    "6erGP3HZPr9Jt74+dxmsP0Iipz9gPdq+vlCtP8XZWL6S88s//GQsQDRVwz/m2kG+clNlvz0p"
    "2779qK6/cvQwv/bdxL7jgKK/XedeP9t0sj8dorQ/Xr42v2V+Zj1p3dA9DlOnvtWaqD76w5y/"
    "jeWLv1NWez/ODZ0+yyJevihisD/JcDA/K9xEvtSf7z51T7u/f1JEP++EBr67eKU+O0bOPmsl"
    "Lj/+eCc/7ZYSQEKvKUAI2f0+712/PvMssj8XcDw/nqPPPifiFb/2vSC/EjaXP9x+TEBkaQRA"
    "1IkxQHwLFD2g7FG++G+svnP9kT/sCRNATjH4P+b/b77JK+U/mXLAP1ghmT++dNA/5+5sP/W0"
    "D0Au8TY/CFp6P2ZLfz9kxEM+b1lYvfk7/b0U1Ca9K92sPQoSBj/LN2k+LOcyv/ldJb7NU/s+"
    "zuH9Pn4FUj+opim+ASmXvpZjYD8vNLY+4EDGvS6IND++w9a+/mXUP2BSzD0f01S/0mEuP/Px"
    "mL6+QKO/69cKPuwrLL+FlVxAd4o4P1A1Oz+Hism/dvSOPmBUmj41Aeq9iX8vQLh1d7+sg7y+"
    "NIDrP8iehL43Ll2/NxR2viu+X77Alqm+BNFYP0mfIL9mdDC/K1WiP6Uuvz3chpo/UJ0JQJ32"
    "JT2cBdq+YjCjP8qUYz+ns888PGQtQBMgKD9F+kxAZSW/vqfkYkDDJARA2tC+vYSuED9rURNA"
    "KPO2P3sIz725oSdAfI0av1jhhz/rOjq/F8oqP79mlr9inSW/GVneP0w6mb5NihQ/m7+Sv3Sk"
    "WUBpAnu+Sm+cv88RkT+0fWs+d6Ofv+w+Q0DN7SFA+S+Zv/mLEEDq/Bw/Fz0JvmgcDUAUTcS+"
    "vyYyPkwkhr8C921AiLaTvk2mEECd9PA/ePY5PwG7SDwHeXK7/Li0PkXnNED5kQM+TqIPv1a5"
    "NT8Icfw+VpcAvzJ+z70gywE/vKukvkk5EkD0Bxs+AAwBvi+nrT4cz3NAxExaQL6XWL6JIUy/"
    "+d3BP1aAOkDx2ao8U1C4vrEQLj37L3u/pAlBP+8KJUBYjWtAkniGQOV4wD/DTZS+eVHcvqao"
    "8T4JjvE/nGakv5wVIb+u5bO+UKMBv2flDz4jbZU/ZUjiP5jJfkBEN1Y//Sq9vWCviD/BRsY/"
    "9b+Kvnb72b5KSJS+0LWZPzVbt70tdSC+NX4EQJeU3D+0vJw+n4Y7QGv+vr41wAI+DhDlP7x5"
    "jD+FCVY/kRiQvkmMJ0Ag2oU/GKrxPygC/j5+57i/pSZ+QM1nir5aQltA51fZPvovYD+biTVA"
    "XAmKP04UpL4tUD09Z4wdQN5AdL99u7q/eFTjPqSXgEBfXaVAnxcFP1Ae6b570p8/+Pplv7Ha"
    "7T8YS5i+1lDUPiYNZD9Ff2++FEbrve/q8D6Olt293BOcPnGOhj++1iFAAKZ2v8HxEUCRHBM/"
    "RfISP+Br7j8Z5jNAQAAyQG7y8766G4A8WTW0Ptzenb5Exkk90Sljv/HQ770+WKE/Wz5Cv8cH"
    "oz2yhKm/gKQLv9uUl79vWKs/8T8DP4uKXj8JRUO/a61tPoROGEBE2SG9roR8QIMJUz8yeWQ/"
    "kQ1hP7nS6D13z+q+GOmpv98FRb94MTRA1/TCPwAwzD+JCn0+eNiivkV6ij6THF09RiiMPwb9"
    "hz47g5Q/qmOIP54zeD4SN0S+dSoZQChHYUA6T++9gDHPv+0IX7+i5oA/C7kHQFLBgz97cTNA"
    "wNxGPz+Sir9P8Hy+xx3OPAdtiD6OqiY/N1naP8+rer/qbYs/5cR8v9qpAEA4S/c+C94AQAOI"
    "mr+7344+uzSev8bY2T87uC4+6YYbv4HHzT9Y8whArvF5v2ZI3j4E12K/hEWsvpeZKjz2SBw/"
    "jR4kQELL27635BG81meJvx59jb8qwYG/XG31P+blsL+zlto/TW+Bvt6e5T9SclO/Sn5fvy3k"
    "8j+qg9o9Tim/vMVzLz0lFL8/8qgRQMtTgr85t7S+dkSPP0NZur5PpYE/W73CPzB5db+imB9A"
    "1cILvxBpcb+f0QG/dkJ5v8qQib8fGIA/PJ6KP+4iMkCUDHQ93gCYPTZIej8aTP6+zTCKv5U4"
    "Kr+FWhpAhO+IP1CBIj83XL6/ir6APe6vjT6wDQg/WUgJPuq3EL+8KI89Ec9Uv6zqY79hr4u+"
    "FOuRv8os7D9u4qy/ya+FvUrhEL4rdJ4/YEC4vii4jD8OmaO/zOShP3ZXyr/lR8m+NaSVP9X2"
    "Db/6IhI/EsCLv/C62D1oQGC/h1Ukv+jVPD8ktCFAU4yCvkl36b6Zryu/nj6qv1BChL49cQY+"
    "Hi1kP0VP6j3Y5bA9hElWP2EQCb9O9ci9T9jTvkq+rL4/k46/Ai8Wvxrj7TxA5rK/XgVdPzVN"
    "R79IBXo/Aih4PocVaj7sGUw+/VF2vqXuAj/xe4C+jVTOPpfI6D8VUtW/gO28PbZ4zb5hsgFA"
    "oHUSQAuQJD/MLxNAlLqQP5jtsD4Vzko/WC8bQPSavz9OyXQ/LclhvhfqJT8n9i5A3eBLQNwF"
    "OEDfVoq9S4bKQM/+iT88GCy/5VXdPrLYgT/Pyx+9mbolPzWRlL4qwvs/lE24Ph42Hr4ejRM/"
    "K+eNP0dDur8EZEa/5uNTPx92yj79+mw+ulD+vkBHoT9iAV9A3DyRPskKhj9/waa/aaOcviL/"
    "uj5qJeA/+sWXvoHSr7yoIbS+OpfNP4hgtL2F6o8/iYD8P1lbyL6s2gNAvIQ0P+jpCb/FDDk/"
    "YSzEvnFZkj/BWqc+g5OVPv/VpD97YmdATl++vugM3T9AVVk/8lKrPDp8oT64NKU+nZghP3rj"
    "vz6J7oS9P54Tv8LHED95Lr0/p1hPv0NHO78iqHU+8z2GPi8iCEC6qSW/ytJ+vbZvHT4pjYU9"
    "s8UoP+FLUL9FUza/V3qpvs5r3r4CmZO/lXU+v/1Snb/urRc/EMwPQCy7bb9xXrI/lOdpv5HH"
    "4r6Krdw+HUbnPbpsSUBnHbE+VU8HP78F570uWuG8L15gP7jahT56ArM+pYquPt+S6z7SAbW+"
    "aJWPvm2xvr6dj7w/u3KsPqaU+z+f6C4+6uxRP60kA7/RP0Q+K/HHPyntlb9/rE4/GavcPvFd"
    "RkCmswdAAfWhP5cHCkDlXwFBfQIAQJNKj7/uvqc92PmJv0wEd73tPVi/08jevxMaTUCY48O+"
    "hF3SQOm0WT4Ey5s/SJ1OPwksPT/6aeU/MAduv6LXIT8HoQs/oPdGv8AA970/sfo9+yufvyA6"
    "LL+wHFo/bCAav3TdEUBio64+ZwZbv9Gsvj9DoAm/IFATv0rlyD9c+4A/NJqKvxJG6D8iS3k+"
    "j+kxQKq+AT++aUy+ovH4P8ANdL+sAiRAhAGzP3xfeL9jjeA/clUWv2oau78Ncoa+i0GwPuqI"
    "lT4wTZI+LxSVP/LkxL/3+w5AcBCLO+Leqz9kxyO/el0Bvyohij+3l0M/felTP3sjWb9Vi8q9"
    "nD0Tv1ExHb4i4do9IPJTP0/viD+kFqS9m8NFvw5YXz/sRdA/E2akvZ6/FL5gUBE/0aqNP7WJ"
    "XkBE5fQ/O7ZkQEybkT+tnFg/IKXcP0L1D7/KoRs/OmArQGfX4D9d/jI/ZKEGPwibsj1NPVW/"
    "mjAYP4F0jT7l1/8+/AAHQKg5Uz6IYZW/5o7Ovd+AGz48pGQ/5Eyfvy54uL6CtHa/b7sLP1Q1"
    "+j8+hwU/hFMnPqcxvj+0KXS/FziWP3gxGEBYi3y/3Ig7QHXeWL8i4vg+8wmvvigCST+3Lw0/"
    "eT2Mv4TafT8bGPE+XfVNQDId/j4Hb+o+Yrf2Px+ZPkCybje/LY6dPjZfBL9ajXE9wDGzvcUS"
    "nb4daiA/q2+Svz5snD2OLEZAHf84QOq7u7/VTXA/MGHjvm+FtD5kp8Y+vPoaQPbReUCZVdU/"
    "pB1EQLWECz8CBoC/Jn43P6hCvL/saEU/kXsZP9S4YECNwnw+L8A5vhyloL97s2JASESSvszs"
    "zL4qypY9ghkNvjAOO0DAwTBABsKmPi2BLL9gzeu+gaIUP+LpyD60uei/VlsuPytFpD8LSvk+"
    "93snv7+tOj/zcom+SpYMv1/0mb6LfZk+cWQOvrcdjj8tR4a+JClmP/7vQ7/eWgJAqDaCP/ok"
    "3T00Zxi/xExqP7Yctb8fnRpANdW0P09MQT6BEJC9DjuWPUDoOb8ZwVE+G5K/v7MDIb6Ml2c/"
    "Np8CQDSSTj+tu3q+vGKxvoHAGj5nG8M7ijr0Ptzy5T/tg+0/BeOHP74z7r8rkCE/3fSWvy9x"
    "9D+eAiW+UBIBP8SnA0DmCWO/fYhKQGf1aD44lhc8Qz46v4jskECU+EY/+cWCvn46Vr8fRQFA"
    "u4C6P4eptL+sndE/F4AxvjcGfL9vBXq9r4Z4PyPxgb68qnY+HKhwv0AvoD/SpQBAb+yUv0TK"
    "ZT8aawc+4rFYvQ6Voz8Zjd0/G8ObPqvkuz/M6hs/1vgPQJBmgz+wAb6+7dTrvm7uiT/E3Fe+"
    "1zijv3/j071EELC/OJjxvJtK4z/c/KhAcuKLv74jl77bPVZAllTRP+AvYb9ae/O+s4KWPx2B"
    "y76QvyQ/cQnFP7Ebrb1jN4W/LUE+v2qTor45NV0/N/pkvxhfRr/egiJAG82rvgn15D+uEZ87"
    "lUbWPgKmfz8OPy++8ACBP1ahUj/Zyog/WFjMPpSKbz/vQ/M+q06fv6KFVj6FysE/cY/zvrIL"
    "lz+nSeo+umgkQBrsKkAp0MU/gNT3Po69Lj+oX6i/ZQ4WP9s9D79Abgi/P6vJPqODX0AwtFQ8"
    "vsVYP2han79l5fK+BsEGP8hwLr9hlCY/WRk5P4njKr+iIxo/WR1Wv3/O4T6wVqM/wGNLQPXv"
    "GjxadoU/v6QYvXSNlz8iPCI/5D11vybMij4aqfo/wEicP9Z5B7/DlNM/O90CvsekMT/Y14Q7"
    "tnIbvx/fiD4WtfU+HcVuPsImdL8lLAg/OjeRPtULPr8ITsY/RgkXv7PKh71FTIS+m9KLv3Sc"
    "Jz8Zy9I/6rE6vvGpYr8Gb4i/Q60gv4H71j75X9S+xjfLPz7u4T5ytiG/XKYDv/qIID6e0MQ+"
    "zVxKP2uCEL/wEHC/sCKVvzojD0AH83K/AFWBP1LlBEA+2T9At6SxQG0SAUD0J4Q/hAcLQMap"
    "sT8cCq0+9LtEQEq+mT/vC0FAk0MUvt47gj+ef3E/PKYevzTm3j8X3C++I7pKP3Cmib/stBe+"
    "6tdkQEg3PD+PqjS884NAP8X4eT9hGL+/70ltv3ywXb7i7mK/p0TkPh4GM0AoyQE/KdAlv80C"
    "lD/45Z4/UHmBv1PiXD/rI38+1Fk7voag474tFkhA1gfYP+Y7Lr5Am2xAwPbIvrHmwD7Wtgw/"
    "dBnXv2UJhT4Pmh2/hCe7vM6jpD5wKQw/uHEfQB6xtr/BNEO+XvSBQNq91L5CSAY/GgmNQO24"
    "GL8sTrW+YulNP4w4Z0AXaPc9I1GCP1vrar+3dEu/9237vgJSiT8L+aY+z1OEP1xRlz+QzeS+"
    "Nj6CP14jyj4GC68+exMZP8obpUAW5wRAzrjwPihPEUBAjzi+kDNCQCNPDj+7kr0/KNgOPw8u"
    "yr7/fv8/SqBKPm88Y0AxurA/cOgAvycR3D85SO6+eEfSviSTAkAtZT4+IprhPFT8Mr4gdzY/"
    "As1YP5/nlr7GxdG8cpWeP2aeWL9uMHJAg4j7O+olQL+U2KY/kFErP/0TRr8AoKS//wEhQAFJ"
    "h0CE+SW/anOVPl0Yjj4f/P8+RC9gP0rK575sJm8/JE8Sv2TmRT4m37++KFXHP4Gf3T8Zt7A/"
    "X35aPmiShz8Uro2/3FEdP+BSFL4/85++I54AQGLdHkAeuvw+R1IqPlWtdr+1T6Y/e8itvg9E"
    "eT6w6X0/Aupev3379j4hPQw+hWTEP33+tz9eoBe/+p2WvwL+gD+MkG4/yWYtPycC3T9fLdc/"
    "CtyNPyR9gr76tHQ/6AtzPziuOz6MEM6+ZEi2PVBLx7/8d0dAcqpnPjLTOEC9wFa/xxfAPjKo"
    "iT+LvwQ/VGu2vn4MCj+5VBc/2ZUUPpDdhz5gGZc/OCREQGl2PD/Ogw0+KgMcPkxtJEASjJQ+"
    "7uzPP/Smmr7DAMW+nD1sPwF3Rj9Ht6w/ZBfFP5g8uj8PdIo/ohcbv6S1ZT/hoUK/YBQYPkiX"
    "9D443cA/aLwWQE5E/j5lHMw+YgVKP1sVX75IvGu+NJlivzw7Qj+spYi+/m9bv8QNy76PfaxA"
    "Upd2P6Z1kr8zgKq+SuelPo7haL/tOWE/Q6gAP0+d1T+XCgZAltGPPyJ+tz8FZr6+uqofP/Vl"
    "or+41oQ/InIXQLnVG76IKTtAm8N9PsQwwj5FOxQ/hzZTPuD25j9csci+pMHXPju7qz4+Ftg+"
    "0j0JQO39yD4uyIO/1AoCv6PTtz+PwF9AwBTFv+qjpr9wbZE/Ou4pQCSokr/O1T8/YnOaP9CC"
    "5j+iP96+zWw/PkQu8L5yGNy+MwvhvfoyRkBTMka/hCKwvigEoz4k9ps+0mNZP/EEjT+Of4Y/"
    "IQZbvmSbiL91vS8+clMFPvdwsj8r4Jo+CH4bP7dPmT8L2VI/KI7WPnKrAr+t2yM/MLonP2eq"
    "s76ixPg9eeOwPn0cg78KEa6+fTEIPm15wD+4Ko4/cOigvSPNFb9vl2++v8fjvJLivz/XfpM9"
    "kivnP8ccQUAck9E/An4Uv9W1QUBcPwK+VooQv/lmgUDvSWM/ynXCP3StcL4hY7C/35ijP3Q2"
    "iz/4tlS/jB6CPwJxmj6WOJ2/T5BMQPQLk7/VQJ0/brF8Phm1Qj/RtQA9L71JP1DCkj9i3SM/"
    "1K48P9SOf7/XyWS/ohEyvwQxur517C+/cM+fP9wS/D8LF1NAVWOBvpuBgL4DsjU/046YP+F2"
    "Ob62/Vk/ZxOJQFgmA79SASZAJbjRvXzlij/4e8A/WByzQBTE6r6WOrC+4lTYPxMzFLuh8xo/"
    "a03Kv1khHr9petM/5/HJP57SeL+yM8+98L4SQAiRuj/bcDC+5rvTvtIDiT8vHEM/TRIgvoSl"
    "QUCx9ZO/w4X3P8JslL1Aldg/qotbQDTS0j7Zsbc+6FeGQAPLg0AdW4a/U+XkPvwkpL86vj1A"
    "EOhRPxiDDb/i0SdADVWFv7NaVz9F48i+k5O3PvIsjj5ETbE/NE2MQE5kgj8AI1C/YOb+viat"
    "Nr+GUy9AsuNJvnRn5j94sVk/f22vPzYfW74FOM89I9mNPhqVID9LDOs/Lk0hQKN9Ib+guN1A"
    "2Tp+v6h+ob9U15S7TTs3PtB7EL6a3QY/ndu0P1zHxj/RFAC+ivHPP8Mlo7/PICA/9VMBwDiL"
    "PUDmpGA/+xl+QGDnF0CaMjK/J04pv3ShlL5UXWM/TMRVv9GTuTzCbCw/6pVfPRgjdb8Dbw5A"
    "BlIxP2AXyT/yfTBAZTYDQNZq0z8J6ms/RxybP6mhQkA+6JS/ObsVPjM7QT+mZnRAPXC5PwRP"
    "ND7CNgRAHqfBP/0/oT41gpQ/ucxFPrezmr/o2bc/1HePP7r16z5XiJK/ifYRQMwdML9Hr2M9"
    "L2UCwNpTMUC98Y0/rn+CPx1+GL4KrTw/snIhQNLFL7/akIy/ENCFPiiEXz2LTAQ+IA2Ev71G"
    "gT7Cvf8+ee6SP3zL874XCsBAdeUbv4Gk1D7YYOQ+4jaUPz6+Fj5m19Y/XlBtPJYXd7/015u+"
    "sI6AQKXGCUCY5sM+sU+lv7BcPD/2ECG/WgHqPzdH/75OekO/0kUYP07uA0DRWSRAEBJDPw9I"
    "Hr/azzu/YwOuP7KYkb/iIz5AoDRtv32yTz4y4eu/ebIHQI6tE74J+RdA+TAhP7tGGkAC1na/"
    "avbkQFgx8b6DRqy+3sB5vxXrnb6RGyo9CYg5v75n9z98+aI/Tx9FPxqFLz/x0No+vhC4P5QA"
    "5T9V0tA/YUUTvnFLAEAejBC/WIk9vh6cdj/sL+0/0nlCP8fSxD+hF8Q+Nq3jPssDkr+NbSy+"
    "iJQVP0ap7T+jcGRAxMF0PoSqpz18njtACt61vifPqz4L9we/7JAPQLvMAT6dXRVAVD9pQL1n"
    "h79p5vw/AJUIQGYw8b3YDP49f4MeQJJ9oD5TvrM+Y7LfvoEJgUBpgC2/G07tPuDmiL85zsO+"
    "OfGdP50mvL38BW4/lo3nvv0Npj6AqJc+PM71P6Wxjr9iGbA/FEa7PnSBUz95PVS+ABEVQNtf"
    "Er4F/Qi/3SY2P1Ddoz3dtcy+6CHnPbQOkL183C6/TcqiP57Dor+sMhM/6mxDP3WuC0C36C8/"
    "xRvcPn4j+771qxu/YIeGPgvoFj7ioXk8kDX1P0nUxj6eZcM+TejeP3MY8T2dy8W/Kz1Qv/Uh"
    "P0DLCIu+WHM1P5p6lb6bJje/oSBlPi8fXL+HdT87/14VPxbZFT9kEg0+ZIkmQL+7HUBLNKE/"
    "pegEPgCLmL12SHw//mcdv+FfLT88T7M+ldjbP286Eb/fNhk/oSGev6jNAb9u586+jU06vzN0"
    "1j/5XilAzKgsvpAvmz+u3rM+Lh9Iv9qhDb5VKR8/TJyNPXyZjD+1yDJANpXCvwzVV0Bq1uW/"
    "nxv6veRxgD9E5ta+tK0MPwjidr8Mn6a/ZW+6P7ZTmj8AXh9AVwk0QBWkSD6skhq/6VfbvrYR"
    "Qz4zYAw/OvzHvlKBVbxi9k87SA1rvzytTb/othQ/LzMlvoU1hUDc7Jm/QNb5Ps756D5uE4A+"
    "aOZaPdQIpUBQk3I/FAiFPpKtG768WCo+o45QPgjq5r6jpq6/O3fHvXLMF0A6MBw/ZwZZvKAs"
    "4r581Jc+GIqpv0+dKkBGrBZA/n02P24dLkCCjL4/CvEXP9BnjUCuKl6/c9d+vxBdTEDi34I+"
    "Mj85v8aKub80Gko+me9Ov/p5xr6+O1xAuK6yP1X0Lb9s8UO/iiTSPoeZC0BSv85AU9a7P8Sd"
    "iD96dXs8CLHnP4rNvL/SLqU/1nluQF6pHD/rFkZASPBsP4Ip0z+T/14+u4IAv/dJGkDU9ydA"
    "UWeLPzQqEj8ivfA/1N5wP/YuOb4S0yNAkrIGQGQDeb3YZVQ+BSOoP3I7qr+M444/CIouQKfw"
    "2D8403xAsXv5P9PTN79r3yA+Wit1P9aeir74ZEJAmpgXPiAXpD/me4Y/vN8hQOaBEz8OrTlA"
    "/TpBQCWx9j+SuNc/wnJyQATmML+eQEm/09GlPwGfAkAsZFy+zICEP5fQir7v4ac/U2ThvqyP"
    "NT//iMs/7lnpvgELBsBQcuQ/hMS4PhNQL0CO3wpAO61Uv2NnUUDbdMw/Hx1lvmgkVj/UjF6+"
    "Pn6kvrgsXT8X12Y/YfKqPp7hEEAXaq8+JJt0vyjjkD+gBEe/FIPyPx5BRT/+e5y+EkbGP4Xj"
    "jD9/hyo/CRjIPyBxD0AOwyO/EYarPtYv6z9BjgA/un/0vQhGf78AsSg/SUiMQEIbQr/7hwQ/"
    "ttYuQPHOIb+T/xBA9j/xPeqVqj5rCEi/NEHHP0UjRr2lYD4/wLEyv5KFYj/agDQ/3zsAQOal"
    "OL59n1tAsvYdP10Iyr3NWmdAHD00QPyyi73u2kdAQKIjP8Pk9D5dRT9AFqM0P5tmTj9QdMW9"
    "UuZOPprOzj/n6Kw/TtkuPSvLBT98GAQ/T5fIPsUYGL5oNSi/CA0PP0+7qjxw9FC/60KRPf0Z"
    "gb6Hum8+X/qMP+bNQz8rcto+yO69P8w4Ij59T2k/pw2+P3vjlr4lgSC+JYzAvLol/j7TWR1A"
    "O2V5PzIciz/mfE4/TnaMPwJxNb4QbFI/+L5QP1rfKL+VUwZAZ4njvvmekr6RdqI/0ZyZPwyN"
    "dr81xUA+8PCqP345KEDhUX5Abq4IQM3M2j/5eARAPuS6v53ooj9txIs/TDdWP06ipj0fJFG+"
    "mg3QvkyZZT/8JWo9m9OrP6b+tT9supm8yYsGPnxvgj+Cpeo/RFh4vuzPJz4OtXo/8B3iPlM5"
    "AUBRbodAd7DTP98Fiz+9Bd8+pgjnPwJn175qRsk/lHl4QEIBiL/k1OU+xz13QJj+gj0k91u+"
    "NNVEP6UbMT7boVK//PoqP9msAj+XGCK/eBMqQF0C9D/OZLS+ajjVvzqsnD5o0BVA3l/GPyzS"
    "lj+HQS09n8CxPxxaoD9ef8o+y2sJv/dvWkCplzE/BboKPihcij7iXJE/IrQTvyMbOb9TlLc/"
    "mvOEv4xhZT/eC3U/V1cNP089BD4Ops493GiJP3agrr63uwS/hqZUP4Lh3T+hGVm/VQ5xP3vW"
    "MD/ABok8xD+Gv6v+P78ahn+/ordRQCitCr8SZ80/EhaQPzfkmUDx6Tm+Uq8CP/+Kzj9TaZy/"
    "bg7HvtzQN7+DVga/PhSUvsdbh76y270+dtqPvkyRjT8Yu0Y/wBDePsal8T+6V2s/eLK/P/Iq"
    "bD+4UAE/b4VxvzYVTUCNG/0+TSWWP/I3MkCAM1e/qpN7vlYPab73qMo/d7onPwLisz8RNTc/"
    "myLMP1Cthz+0Fk4+9mG9P4MNEEAFeCO/UKCLP4jMU0ACtoo+I3q8PunvMr8lDs0+nJidvk7i"
    "b0CKjvM+e8+gP3y5Jz0uEqo90/N7Pw4djr+FTtS+YE2Ov2i7Ez462N8/AWB5PoHQeb73YHA+"
    "JZybv1f76j+fL48/jCpOvm4KSr/WdBlA6rOOvyzkPb1QsYQ/IkqZP/Yt/r54Br4/t5hRQPzb"
    "hL5PHEBA1vZfPuAwzz4npJW+LvnYviRvk78rlT0/Zye7vvpHqLwOn22/wWQ8v/5XDUBERJpA"
    "38lhP4xHrj+P7vc920knQH9eMr8xWr88O5AsP7e5hz+lsTO/GTiyvCNfLD9e3KlAsOeIPhSo"
    "JL8JAbo+ohUiPfqjhT+7HZc/73h9QKXQkD+1f0q/zGQfPlZ5a0C8JfY/Cwk1PpRn5z9cX0k/"
    "XJMHP5z2zz+qII+/QPWevpkizz9r0KhAuBB7vsSNFUD6/0Y/BgxTQLGwEj6mfMA+Q9usP4vn"
    "/j8P1TM/au20vSIjwb9gWoW/n0dnPpkyCkBi3Su/vETZP93APrtEIPM+ChWhPnRZib6j1j4/"
    "/H/0viuPEEDOoe4/BIGLP+bnvECStXq/xyUAP9rW5j+EYRc/hKINvzU5Uj+MZRFATF9Hv3J7"
    "Az7s1Is/bpb2vWVTor4ZbWQ/WumvvyA8/D6UPS4+lFBIv7q+ib7LPGZAoUZEPWGgl77PrQRA"
    "p7DTvnB0nL/Yhz8/nojOv7TqUECfqZA/D0eZvxeiEb9mxX69guvUPh5dB0CmOvQ9dsZMPQ6y"
    "OT8/2Kk/yygyP88fGb+mxuE/mFxev7LPB7+6BSi/QxvXvlEwdDu05sa+eDxYQAyi2r5RkQk/"
    "6NH1vnybpz+wuO4+AgWLP8xzNj9jK2m+xg1+P/mmQEAB+Q++rCOPP6zYkL5mn5Q+ic8fvRiU"
    "9j2v2Tq/8fV9QD2gSD+E8Vi/VCSHv9aQ1kCN6Rc/p0uuPs5Ttr0eWlG/ylPpvnaLST78TO0+"
    "BpwMPlwgYb9Dy4e+/pezP+J6wb/3qL2/d3bEvg86B75M/oo/qMfoPzjXR79/Z2Q/0VEuQCKO"
    "8D+15jG/1iA1PuaGwz9Ap9Q+amSjPwUWxz5UM1A/dPPOPt5GhzyzAF28/kMUv7Dumj7QsYQ+"
    "ejRxPl3RNz9lN6A/fQUKvxWBOD9yO46/6/UvP+LxiT8Smuk/1w8lv8uw+z/MCl8/1u6APsXF"
    "tT8eOV0/JMoFvwvCsD6VhIy9+Z0fPyTyv71wzku+OHlOv4rtyD9PWu4/ZKKXvzDS+D6CUyxA"
    "QVGKP4vUA0CsmiO++SoJQAlwXT8ASd4+a+K3PtQuWD/zNpFAaPqHvwMCur6YCNs/89VmO+2u"
    "JDtatRy/1nGbPhSepD/nph0/MOreP94YLkB3By6+CGXCPaw11r9s9WS/i84UPwd9q711jgS+"
    "2QzoO7pMdr4On/M+8g/GP/iPOr/OxS4/smvDPoD8v739Ae0+ZPBuP7AOxj7s9C6/aDT/P1Kl"
    "sz+IqAu+Bh50vswTxz6b/8s//7cGPxLmqj1LvpI/ruyFPvauLECGvXlA+XQfQDU2ID/24v29"
    "m5kYPiHiqz9Qc9Q+yW2/P8q0REDo7qS/vTDRvopkAT/QQqw/H5CfPgytij7ErTQ/Qjz3P8pL"
    "1z5Zf2E/WM8FP6rMPr8QiIo99GiwPgApuz5b/RlA7S2CP1CwU77cMsW+iZAMQG7WSD+Bq1a/"
    "jnpfPqWwaL5y0PM/VbglP0I3WL5RdjRAcGlPvhQGTr8sZoG/2DYvQDiUB0CaJCw/+L+iPyKK"
    "7D/UIRhAGSKbQEajDr4EwLy9nId4v9aJjT/kv409JTiwv6VVFL9OdEc/SPcBv6xgJ74fXL4/"
    "FRSKv68IbUCRBqw/etH/vphSPEDZj8Y8AVRAvUTN3L6iJ6U+AKBxP0Ho0r41yMI9nb3dPqQw"
    "Oj7ve8K/OZX7voZfj77tf4M+sUecPlwfCr8cuhq/3Lgvv/R/Fr86Vwm/rG0AP+KPDj6u7Zc+"
    "LmSBPtEkjD/qB7K/XLn9vtmhZD+7QKa/wumgvJ20jb5u0ik/8dh4vjOuhUDTMpU/McM7PYBu"
    "BD8GkJg+vRqLvz37zb7lolm+zCKqP9RONEBZkjU/S3Miv/kMqL8LqZw85ejwvcRxzL5iR4g/"
    "dZYXv+SZST9XeUC/DlqZQPKycT49bfc9s3gVQPJpMb97nJc/JHdiv9tvsj+bOOQ/bOAUvxha"
    "br6KXRw/LNVrP+LRg79vEre+nEqPPxj6EUC0uIA/AMPYP6zqJz81+Ek+bRa1vjqy3z+6aGlA"
    "YKNiP9ZJwz/QolE/bWkMP7Oabj4dicm/YrEkv1raVb8fbZU9Fb64PccWab/jTcs9fnXvPsWH"
    "kz+VFGO/LW/5vx8IcT4S1iA/uYKDQA4shD5V8dE+Q1gTP6T6zr0IZZ+/C+KDP0O8Uj4wwpNA"
    "BYBPv29zhT98E7xAozCJv9OKLj8ahWi/OxtyP4pGIEBzl/g/cXTWvkyt3zxEv42/QC0oQLTD"
    "8L52hZQ/bty2vndD5r2Kmly/8v3bvfJIsz/KN5Q+J+4NQASgwj+WvSm+60gOvq3Pvb7b92S/"
    "bcQOQOCk/j+nqVK/jtyav/gzG75dm3s+UFnZvUA1SD814uc9L8IaQCidqr1+r44/JYetvv1/"
    "AkAACCNAThb9P+wBob9M+30/wgC+PmK8zj/jVSo/M8Y3v67Cjz2+kHe/Z5lGv8G9wz1sBx2+"
    "rfg2v/f1tj4jA+I9EIWAvJj+Mj8cpxi/s8SnvK81ob9uPVdAXbWBP4yRyr45uw89Zvq1Pmoh"
    "WT8w4T6+dv0MQKyQ7j7NnRhAgs5Iv3hTJEBeeMU/ueYOP8FYfzymUf2+5JDpPtktWL4udW6/"
    "CED7P69K9L7YVnc/rZ3sPb5Hnz8kaSA/glViP+oph75e9e6+LcPXP2dlaL+fqrS/PG4jvxpY"
    "Wr0IUnk/cvzRPot3qj/CtlC/xKvJvL4rAL9WQhJAUyz/P/QK179E2Qu/Z8WpP/QEDD9BLk8/"
    "/2uJv55P0T9+xdw/D3/YPlB9Lb/yFeI929LXPzkdqD85bAO/GA8pvyFPSr87ek0/u0/SPFl6"
    "Ez2iTMI9ewhuP3ZBOr9ap3W+okaKPwMU5j+IfXY+z9V+v09pyD/3dtK9Hg28P2bXeL8W2si+"
    "J2O2PrJluz/U3C5ADX5TP8Rr7DyyMYI/EE0LPhdc4D52M2Q+cPc7PwiHvD4ip4E/nlQ6P5vP"
    "WT/P0Pg/NGSFv0z+gz+BNg9A/aUcvqpzpT/SQTo/OheJvrErAkBbgpY/flY3v0wh6j5Sso89"
    "si16Pk5dmj3LJjdAosd/P0Y4R79esZu/sDi+Pygptz9UJdI/ncSavjUWub78jXy/bu+kv7EX"
    "GT+fAMU9Vq5uQLUv4L7eBx4/itCAv22C5r9R4KY/v8Kev8CTYr9h/EQ/HJ86QGZg5j8kmUa+"
    "wn2Cvs79P76axiM+BB1jP7tVjj4n6Sc/6uJdvl2MKkBGV04/CjlsQCyPBL9XfF8/jIWXP7HO"
    "CUDKCwG/JTRrPgolbT5mxEi/Sjzkvgllwz488+u+SML0PyaCB76ROhO/LJtvPp5Dtj+kh70+"
    "xFF3P3WLXr8DpUk/IsZwv2ezmj4wlClAjOQDQAW2yz2vLRe/nKw5PwX4sj996kJABon9PwXF"
    "mD8GUfy+znQNPhySCr9C1v2+/hCpvsHM3T9h25k/UXS8P1iUxT8y+mk+IOPbvm/Xhj4DbGC/"
    "Uk4Xv9hzm75XiRFAgyseP8NvRT8mJgW/IwLevktnLj+HAi+/S9Uhv6Xfgr+sw6A+Ara5PlnH"
    "8T/3nmo/RCUSP8niOb+Itk0+k65rv2KCY0BnIQy+tnnwPkMXRr8Hmao/Y6Q1vi6YHEBqFle/"
    "FmjIPs6p0T/2VIi/UPTOPmQpUkAH+l5A2UyJPpgH0z/IDipA0TTYPn/xf76KAPW+2cmHQCWR"
    "rT4EqwO/LZZMP4f70D62Oxw/PVZ7PSlGKz8SUYC/tg9dv5GsKL+x7gdAtaEKvWp4VDz67Rq9"
    "mo50v1PgPb/DVam+CAUWQAJRvz8lmxRAbtTZPzgunD6H8jI9qV+wPTiO9T7USSs/dLn7PmUw"
    "Xj5lnbc/Xt9nvzDoVT8d26G+COTyvkEcjz5IuiY/UNnXvrUY0z8L8CK+2I8Xv5mhlDxDKs8/"
    "+AZ/P6qHtr5YyC+/kKgUPljtnkD3gkc/AasrQAVXGz/rrs4+N0rSP7eBvL2s6qK+ZQ2WvwTP"
    "hj/QwRU/GkGtPpOGrj+unW6/NED3vv8yDr9K6MO/IeiOP4gR877G6TG/RSKhv9hsdT8bfky/"
    "hlOGvN7PFj8kEQI/h6saQOM2H7+mFUNArN4VPr7pbD5YjLs/4uF3vpAzWT9UXso+MWALP0sm"
    "0T5SsEc/c+o6P4N8Mr/wtyZAuZWqv8hYDj+PjIu+GNi/PvfNHEC/EWe/eDlGviXPCz+86ia+"
    "vbW9vk46gUB5M5M+H1MmvpnNekAguGY/Pv8qvyuaZUAsBbQ/qIOsvreKAL1AUWC/A0SzPn3a"
    "3r32v0tAmsNcQHCoBL9Imsc/hxmhvzM96L1o/pW+I9uoPw3sjj5uI8w/8ALTP9eRW79LagNA"
    "vbzJP+aooD4tBIM+RzB1vUN9Jj9GXTs/JCMQQKmYWrrWCOy+8LJ/P7/qi78M+ho+mtyEv0Ar"
    "V7+0x10/cPdTP1sDJb+e1ku/z5upPpYOZT8mAOG/JM27Ph9DCL8ror6+VJB8v5V3Yb/fSa+/"
    "9wkYP+Heqz7bQps/w2ETP3yf/z53Nkq/21pbPgBt7D8b+xE+wq5PP2x8hz/QUwG/vRJHP5La"
    "QD+B80m/D0e7P+UZpD9I9ry+XltXP8fZgr/4FI6/EnODv6G55j2UGCs/fQTSvuaQ4T7Pvo49"
    "6GwCPnDsCEAKDZK/SZ4GPt9LpL/1bgZAnnZEv9UCDD9f/cA/AJl5Pv/cs70GALk+5Vp/vj8k"
    "rj9w0B1AkkFfv1we3LxqHUQ+tGEqP5Zbl7/GpfG9jvwdQJpbJj7hDW0+v9k2vjiOfD/jzGy/"
    "ideEvvboob/FEzS/Do9TPyKyJkDGvIM/A5UBv7C0F79YEks/Q5wNPZKqM74IiZi/ifwwPiBn"
    "xD9L+ZtAAL81vktAEkCBkJk9j0lYv7TjBUDuWkdABKY1PsBdJ79eLTy/zuKTP92amr8n8mm+"
    "obUsvrSdpECvU8m+5G7pPLUDwr/qjxo/dqIgv1DXgb4FJbw+kosnP/S6gj+i7o4/d7L4P76s"
    "3r4b6uE/qgjTvgi2F7+7QJo/7o+5P3ICnT8aXzE+aZ3ZPTzbFb/0QUG/+9rdvZn6QD8+X/o/"
    "5ZRLP10VbUAECF2/F6+XPiJoxL39OkE/TG84vwUuJ79+14q9EE+jP3+ng7/YjaG+QCAdPyJE"
    "zTyNNaw+Y3d9voHd/j2mDdM//fg1QHe7hUBH6cw+BulHPhpp7b5Ux6c/+aZtPqksvT6yaKA+"
    "UvMnP7Toj77UdRBAk3+mP2yqSj402m6/PDkWvu0TIj6LnZY/Dby3PvG1CkCKIwo/Lr54v5R3"
    "0D8qxxu/tp2CvoHxQL9L/ce+TkRTP+0BGr0mP46/YleIQEYmeD/FGGo/9XyQP33Hkz8ZrWE9"
    "AJ3tPq58jL7dU1o/IWjWvgyWVr+6dK5A8qt9v3A+/z92UW0+hdGIP2Uhkj/11nE+SrP3vsGp"
    "fr/ZrmW/EwypvlT+dT7QKwQ/8t1pQAJENEAvKJw+jDcBQE1F3D7n1Je+u3rnvwMYGr5zt/a+"
    "uGYOv3kYvbykK5g//uaAv/d/e0B0Z7g/7iIzP6wVJz8ZxRa/7e2OP/hmO0AaJk6/kydjPVXi"
    "xz66g8k/lXSIPGvlUL9na569U51EQIYo/r1+RkA/b00DPwYyZkD4u9Y/SSOFviKhfkDUiMa+"
    "X1o6vf2ZFkDciaI+FL9ov+88xT6f1ElAflWnvl7l+D6NIfM/asw2P97lf7+rbJs/v3W7P0p6"
    "a76nn1Q/YIZav91oAz56VCm/o1/2PsSCNz/VfZM/aVFHQPyYaT/7hYo/QHHGPu3HE0CsBCs/"
    "bR2+QM++KED2NXc/Bm91PyF3Cj9AM5c/dfqavp0V8L5PDie+W1UnP5qtkD65y0y/83sEv9vK"
    "xD4Q4KQ+OtDFP4utwzzD8LE+hGt4PwDG5b5yqFU9oQluP07Atz/Ah5c//nQov8PDDL3qwWQ9"
    "VW1MQJoCAL/blYQ/yMopv+qLjD+2QUG/ZB4QP8qFhL/Rp6I9mSv/P+sZyT/BeqY/yYPiPjp1"
    "6b4Yw8w+ovUDPoGRO0ByUKU/Ot2UPxqVLECRIWNAkzdavjYmBr66DO0/cc4rv5CbMUCH9aA/"
    "D8IaP4mOgL+la2A/EdzQvmHRWUBtW3i/wMZqP+nTvj8eNg2/rujNP9vgtD8aCSY+gLn1P50A"
    "iT7LNg+/3EL4PqnCZb1FTTq+yJfhPuzp2D4mXrE/9OvUvBEIhr/+dZM/8jukPyk6UL8Atwk/"
    "OGYWvyBTkD7R4Ei/VQkKQMr71T+HM35AuLkcv3WYVb869Qi/zp/JPzneMz91e6I+NC8OQBJe"
    "gj/jGAm/Sr4MP6McrT/JCkq/alKwvnP1wz91rTo/kd+svlK34kCLClA/4EpXv4+gZb96wKpA"
    "5ZjKP/wjsT4+b5u+A7oyPygJNrxLd6o+XoeQvpYBTz3k6sM/QL3pP91Urj7Mcf88/8q5PkyG"
    "Fr7PgBo/sAOIPiFKkD8pjT2+ugISPp3dO74/YxG/ckmhPxFPDj+QcU9A7S2DPmS1xT8DJ9w/"
    "2kD9vuHPmz9hCbw/OU2bvuWeZz9wRgVANMLGP3Mlsz+dtI2+pQyxPqV21z5q0pK/VSHEvtgC"
    "f778bjg/MhnsPhI2G0B22Sk/5QauPsWYr76LRFK+PWqTvi+BD79gdmy/6pPqP2ZVWL8IHew8"
    "1X19v6h8aUDd+P49FGhcv4osQ0DKXQE/iJUfvlTgwT7xc3y/ZytyvjeihEB0//C9fazBP5yO"
    "kb6kgi1AW1/hvVr/9T6ISR4/97ogP/QDcL9wb54+sPSdPwAeI0DRIdE+uno+v7qxjr9CE/0/"
    "4bTqPwe/U7/HMYK9RdtfP3TDFEAJNy+9F36nQEWBY75pd/0/cUhcP545QL5fdw1AWzWHv8ka"
    "l0Bo5Ic+xqcHve8GyD1n/t4/pCnnPPgHNT/RcbY/2pboPh5sgz/B13U/WyqMPoiiab+Bqic9"
    "hFlBP0AzKr5wqDC/XOYxPHUUx76yvARAHYiav9YUTL+km6I+xqLsPnX4rD6Z/3c+Hic3PvQd"
    "ar94cek/oZe1PzZJZb89pq++GNeBvpJEYz+WRaY/cQVCQOSxGD8axu8+EtzCPv12wD969HQ/"
    "vfGJvzyMp76DZ8g/kf47vjFkIL/eZSQ/mkUGP33imj9EKlA+FYnmPaVlbEDphMM/lFEUvja2"
    "Fb4sBQ5Aqtmgv+8hkb9ms7E+3xKmvigJyL6jiBw/N/CoPyg7eT8dpzm/WIxTP4FXPkB5mPq+"
    "lQlQvlbYJr+3BxY/YOxRQEGmGj/DVvO+rrPIPwvrIUCjmJK/kj5hPwLhmb/s4PU+LT8OvaxV"
    "pz5EMj6/dSzYvhqJmr6+5cQ+ZwhbvlCzJz8yZ3G/supoQIH3xz7dDdU+I9WwPprUSb/yYJ6/"
    "4KSkv7ivOkBADaS/xI5jP8vtD79wQnw/dm16P1pTDD+85ug/Zm3HPyIiq782wrw+RWCCP5fK"
    "T79FqIY/204Iv039m7/ncs4/rTTbvpO6PT80/w0/m+y8PwUaBj0kgQC9CQpGP9bVAz+Wq3U/"
    "oa4NQJPCnLzKMUY+vvDsPdU6BUDsUU8/OlJgQJs30z/Itw8+5lI+PPPNNj73vzNACgZiPy8+"
    "db7qTk9A9FUevjRcJb+ivGw/wy6bv/sMrr0="
)


def _constants():
    h = _hamming_ball_np(_BLOCK_SIZE, min(_HAMMING_DIST, _BLOCK_SIZE))
    n_ball = h.shape[0]
    h_pad = np.zeros((_NB, _KP), np.float32)
    h_pad[:n_ball, :_BLOCK_SIZE] = h

    changes_pad = np.zeros((_BATCH, _KP), np.float32)
    changes_pad[:, :_BLOCK_SIZE] = h[_CHOSEN]

    g = np.frombuffer(base64.b64decode("".join(_G_B64)),
                      dtype="<f4").reshape(_BATCH, n_ball)
    g_pad = np.full((_BATCH, _NB), -1e30, np.float32)
    g_pad[:, :n_ball] = g
    return h_pad, changes_pad, g_pad


_H_PAD, _CHANGES_PAD, _G_PAD = _constants()
_CONSTS = np.zeros((256, _KP), np.float32)
_CONSTS[0:64] = _H_PAD
_CONSTS[64:128] = _H_PAD
_CONSTS[64:128, _KP - 2] = 1.0
_CONSTS[64:128, _KP - 1] = 1.0
_CONSTS[128:192] = _CHANGES_PAD
_CONSTS[192:256, :_NB] = _G_PAD
_CONSTS[192:256, _NB:] = -1e30


def _sampler_kernel(x_ref, wt_ref, w2t_ref, c_ref, out_ref):
    x = x_ref[...]
    wt = wt_ref[...]
    w2t = w2t_ref[...]
    h_sel = c_ref[0:_NB, :]
    h_ext = c_ref[_NB:2 * _NB, :].astype(jnp.bfloat16)
    changes = c_ref[2 * _NB:3 * _NB, :]
    g = c_ref[3 * _NB:4 * _NB, :_NB]

    xk = x[:, :_KP]
    ub = changes * (1.0 - xk) + (1.0 - changes) * xk
    s = 1.0 - 2.0 * ub

    def _nt(a, b):
        return jax.lax.dot_general(
            a, b, (((1,), (1,)), ((), ())),
            preferred_element_type=jnp.float32)

    x16 = x.astype(jnp.bfloat16)
    wt_hi = wt.astype(jnp.bfloat16)
    wt_lo = (wt - wt_hi.astype(jnp.float32)).astype(jnp.bfloat16)
    dxk16 = (ub - xk).astype(jnp.bfloat16)

    base = (_nt(x16, wt_hi) + _nt(x16, wt_lo)
            + _nt(dxk16, wt_hi[:, :_KP]) + _nt(dxk16, wt_lo[:, :_KP]))
    base_hi = base.astype(jnp.bfloat16)
    base_lo = (base - base_hi.astype(jnp.float32)).astype(jnp.bfloat16)

    s16 = s.astype(jnp.bfloat16)
    wtk16 = wt_hi[:, :_KP]
    kidx = jax.lax.broadcasted_iota(jnp.int32, (_BATCH, _HIDDEN, _KP), 2)
    a3 = jnp.where(kidx == _KP - 2, base_hi[:, :, None],
                   jnp.where(kidx == _KP - 1, base_lo[:, :, None],
                             s16[:, None, :] * wtk16[None, :, :]))
    act = jnp.maximum(_nt(a3.reshape(_BATCH * _HIDDEN, _KP), h_ext), 0.0)
    act16 = act.astype(jnp.bfloat16)

    ridx = jax.lax.broadcasted_iota(jnp.int32, (_BATCH, _BATCH * _HIDDEN), 0)
    cidx = jax.lax.broadcasted_iota(jnp.int32, (_BATCH, _BATCH * _HIDDEN), 1)
    w2b = jnp.broadcast_to(w2t, (_BATCH, _BATCH * _HIDDEN))
    m_mat = jnp.where(cidx // _HIDDEN == ridx, w2b,
                      0.0).astype(jnp.bfloat16)
    logits = jnp.dot(m_mat, act16, preferred_element_type=jnp.float32)

    score = logits + g
    m = jnp.max(score, axis=1, keepdims=True)
    jidx = jax.lax.broadcasted_iota(jnp.int32, (_BATCH, _NB), 1)
    first = jnp.min(jnp.where(score == m, jidx, _NB), axis=1,
                    keepdims=True)
    onehot = (jidx == first).astype(jnp.float32)

    hc = jnp.dot(onehot, h_sel, preferred_element_type=jnp.float32)
    out_block = ub + hc * s
    out_ref[:, :_KP] = out_block
    out_ref[:, _KP:] = x[:, _KP:]


def kernel(x, W1, w2):
    return pl.pallas_call(
        _sampler_kernel,
        out_shape=jax.ShapeDtypeStruct((_BATCH, _DIM), jnp.float32),
    )(x, W1.T, jnp.tile(w2, _BATCH).reshape(1, _BATCH * _HIDDEN), _CONSTS)


# speedup vs baseline: 17.9598x; 1.0369x over previous
import base64
import itertools

import jax
import jax.numpy as jnp
import numpy as np
from jax.experimental import pallas as pl
from jax.experimental.pallas import tpu as pltpu

_DIM = 4096
_BLOCK_SIZE = 10
_HAMMING_DIST = 2
_BATCH = 64
_HIDDEN = 64
_KP = 128
_NB = 64


def _hamming_ball_np(n, k):
    ball = [np.zeros((n,))]
    for i in range(1, k + 1):
        for tup in itertools.combinations(range(n), i):
            vec = np.zeros((n,))
            vec[list(tup)] = 1.0
            ball.append(vec)
    return np.stack(ball).astype(np.float32)


_CHOSEN = np.array([
    36, 51, 51, 33, 14, 35, 52, 53, 28, 27, 28, 53, 50, 11, 19, 0,
    3, 31, 51, 11, 37, 41, 0, 11, 23, 13, 15, 36, 20, 25, 44, 51,
    16, 47, 27, 28, 12, 17, 25, 29, 6, 8, 50, 34, 8, 33, 18, 40,
    1, 8, 23, 23, 11, 31, 19, 32, 47, 21, 40, 53, 48, 43, 32, 20,
], np.int32)

_G_B64 = (
    "Ia+SP3WDtz8+qAi/07OUvjHsSb+Zeu++yigMvt1pT7/MQS0+F+yePlHzmb3iaWm9TJsTQKzZ"
    "Hz9wjGA/zLpKPe/QFL88gW0/x03UvXXMDL8Nm4s9McSgP+MhF74SDYE/9wRTPltj7T7Ijqs/"
    "2KQMQBy+uD9dPzO/s+WpvsNhhb+vbQ0934VaP5M0Ar+o9My/PGvSP1mwD78K5XQ/rgBpvkTB"
    "pj5H15s/uHyovo0H8j177Bi/kRuxvwEkWEDvTVU/NlvIP/s2aT3iw349FncCQDw0ID/Xpk+9"
    "7snzvRRr/z+PYGi+7S+xPxgHFL8M0so+XT5LP00NDL/G+jU/M6lpv9ErnD+VkE+/WDnrPvV+"
    "eL5FZC9AF3QkvrC5j77zzqk+23fMvlq9FT8YgQ1Af2rcPpQHQz/Mzea9MDs/vwulWj+Vsxy+"
    "cKdoPrIIU7+MCiW/fAQfQH/CcL/CJXg/AGRBP7BsMT+G4rI/aGjovqDm8z9LRyY+TEnCPn1B"
    "grzSCDtAqhR3Pjgs5j5qoQdAdB2Vv1bq6z8170i/5uFAv1joAEAwvThAZPCCP/nHmD6xeAtA"
    "4Y4qv07jHkBR2M6+2C0Nv7yGg7+//7s/rT9Yv2eatz8dt6o/UQ5ov5ZyAL45MNw/BDXiPyYE"
    "479g2+k/phATvyitqj9uobY/+Vy1P6HOBUA7/5k+ydIkv2AQyr+kzOE+JJxAP4Fumj+fQhhA"
    "ylqGPhwPDkAqsHc9ZmmKPWS1ob+qT0M+5Ag0QIeQBr8dRy5A5yORvxiolD9mFbM/ZX3Zvfwh"
    "vb/fdjJAMI03P9do3D9NjY2/2nSqvnz7fz5nxsu+EQ/7vuvJPL+Gy84+tHs9PYF+jz4hqqE/"
    "6erGP3HZPr9Jt74+dxmsP0Iipz9gPdq+vlCtP8XZWL6S88s//GQsQDRVwz/m2kG+clNlvz0p"
    "2779qK6/cvQwv/bdxL7jgKK/XedeP9t0sj8dorQ/Xr42v2V+Zj1p3dA9DlOnvtWaqD76w5y/"
    "jeWLv1NWez/ODZ0+yyJevihisD/JcDA/K9xEvtSf7z51T7u/f1JEP++EBr67eKU+O0bOPmsl"
    "Lj/+eCc/7ZYSQEKvKUAI2f0+712/PvMssj8XcDw/nqPPPifiFb/2vSC/EjaXP9x+TEBkaQRA"
    "1IkxQHwLFD2g7FG++G+svnP9kT/sCRNATjH4P+b/b77JK+U/mXLAP1ghmT++dNA/5+5sP/W0"
    "D0Au8TY/CFp6P2ZLfz9kxEM+b1lYvfk7/b0U1Ca9K92sPQoSBj/LN2k+LOcyv/ldJb7NU/s+"
    "zuH9Pn4FUj+opim+ASmXvpZjYD8vNLY+4EDGvS6IND++w9a+/mXUP2BSzD0f01S/0mEuP/Px"
    "mL6+QKO/69cKPuwrLL+FlVxAd4o4P1A1Oz+Hism/dvSOPmBUmj41Aeq9iX8vQLh1d7+sg7y+"
    "NIDrP8iehL43Ll2/NxR2viu+X77Alqm+BNFYP0mfIL9mdDC/K1WiP6Uuvz3chpo/UJ0JQJ32"
    "JT2cBdq+YjCjP8qUYz+ns888PGQtQBMgKD9F+kxAZSW/vqfkYkDDJARA2tC+vYSuED9rURNA"
    "KPO2P3sIz725oSdAfI0av1jhhz/rOjq/F8oqP79mlr9inSW/GVneP0w6mb5NihQ/m7+Sv3Sk"
    "WUBpAnu+Sm+cv88RkT+0fWs+d6Ofv+w+Q0DN7SFA+S+Zv/mLEEDq/Bw/Fz0JvmgcDUAUTcS+"
    "vyYyPkwkhr8C921AiLaTvk2mEECd9PA/ePY5PwG7SDwHeXK7/Li0PkXnNED5kQM+TqIPv1a5"
    "NT8Icfw+VpcAvzJ+z70gywE/vKukvkk5EkD0Bxs+AAwBvi+nrT4cz3NAxExaQL6XWL6JIUy/"
    "+d3BP1aAOkDx2ao8U1C4vrEQLj37L3u/pAlBP+8KJUBYjWtAkniGQOV4wD/DTZS+eVHcvqao"
    "8T4JjvE/nGakv5wVIb+u5bO+UKMBv2flDz4jbZU/ZUjiP5jJfkBEN1Y//Sq9vWCviD/BRsY/"
    "9b+Kvnb72b5KSJS+0LWZPzVbt70tdSC+NX4EQJeU3D+0vJw+n4Y7QGv+vr41wAI+DhDlP7x5"
    "jD+FCVY/kRiQvkmMJ0Ag2oU/GKrxPygC/j5+57i/pSZ+QM1nir5aQltA51fZPvovYD+biTVA"
    "XAmKP04UpL4tUD09Z4wdQN5AdL99u7q/eFTjPqSXgEBfXaVAnxcFP1Ae6b570p8/+Pplv7Ha"
    "7T8YS5i+1lDUPiYNZD9Ff2++FEbrve/q8D6Olt293BOcPnGOhj++1iFAAKZ2v8HxEUCRHBM/"
    "RfISP+Br7j8Z5jNAQAAyQG7y8766G4A8WTW0Ptzenb5Exkk90Sljv/HQ770+WKE/Wz5Cv8cH"
    "oz2yhKm/gKQLv9uUl79vWKs/8T8DP4uKXj8JRUO/a61tPoROGEBE2SG9roR8QIMJUz8yeWQ/"
    "kQ1hP7nS6D13z+q+GOmpv98FRb94MTRA1/TCPwAwzD+JCn0+eNiivkV6ij6THF09RiiMPwb9"
    "hz47g5Q/qmOIP54zeD4SN0S+dSoZQChHYUA6T++9gDHPv+0IX7+i5oA/C7kHQFLBgz97cTNA"
    "wNxGPz+Sir9P8Hy+xx3OPAdtiD6OqiY/N1naP8+rer/qbYs/5cR8v9qpAEA4S/c+C94AQAOI"
    "mr+7344+uzSev8bY2T87uC4+6YYbv4HHzT9Y8whArvF5v2ZI3j4E12K/hEWsvpeZKjz2SBw/"
    "jR4kQELL27635BG81meJvx59jb8qwYG/XG31P+blsL+zlto/TW+Bvt6e5T9SclO/Sn5fvy3k"
    "8j+qg9o9Tim/vMVzLz0lFL8/8qgRQMtTgr85t7S+dkSPP0NZur5PpYE/W73CPzB5db+imB9A"
    "1cILvxBpcb+f0QG/dkJ5v8qQib8fGIA/PJ6KP+4iMkCUDHQ93gCYPTZIej8aTP6+zTCKv5U4"
    "Kr+FWhpAhO+IP1CBIj83XL6/ir6APe6vjT6wDQg/WUgJPuq3EL+8KI89Ec9Uv6zqY79hr4u+"
    "FOuRv8os7D9u4qy/ya+FvUrhEL4rdJ4/YEC4vii4jD8OmaO/zOShP3ZXyr/lR8m+NaSVP9X2"
    "Db/6IhI/EsCLv/C62D1oQGC/h1Ukv+jVPD8ktCFAU4yCvkl36b6Zryu/nj6qv1BChL49cQY+"
    "Hi1kP0VP6j3Y5bA9hElWP2EQCb9O9ci9T9jTvkq+rL4/k46/Ai8Wvxrj7TxA5rK/XgVdPzVN"
    "R79IBXo/Aih4PocVaj7sGUw+/VF2vqXuAj/xe4C+jVTOPpfI6D8VUtW/gO28PbZ4zb5hsgFA"
    "oHUSQAuQJD/MLxNAlLqQP5jtsD4Vzko/WC8bQPSavz9OyXQ/LclhvhfqJT8n9i5A3eBLQNwF"
    "OEDfVoq9S4bKQM/+iT88GCy/5VXdPrLYgT/Pyx+9mbolPzWRlL4qwvs/lE24Ph42Hr4ejRM/"
    "K+eNP0dDur8EZEa/5uNTPx92yj79+mw+ulD+vkBHoT9iAV9A3DyRPskKhj9/waa/aaOcviL/"
    "uj5qJeA/+sWXvoHSr7yoIbS+OpfNP4hgtL2F6o8/iYD8P1lbyL6s2gNAvIQ0P+jpCb/FDDk/"
    "YSzEvnFZkj/BWqc+g5OVPv/VpD97YmdATl++vugM3T9AVVk/8lKrPDp8oT64NKU+nZghP3rj"
    "vz6J7oS9P54Tv8LHED95Lr0/p1hPv0NHO78iqHU+8z2GPi8iCEC6qSW/ytJ+vbZvHT4pjYU9"
    "s8UoP+FLUL9FUza/V3qpvs5r3r4CmZO/lXU+v/1Snb/urRc/EMwPQCy7bb9xXrI/lOdpv5HH"
    "4r6Krdw+HUbnPbpsSUBnHbE+VU8HP78F570uWuG8L15gP7jahT56ArM+pYquPt+S6z7SAbW+"
    "aJWPvm2xvr6dj7w/u3KsPqaU+z+f6C4+6uxRP60kA7/RP0Q+K/HHPyntlb9/rE4/GavcPvFd"
    "RkCmswdAAfWhP5cHCkDlXwFBfQIAQJNKj7/uvqc92PmJv0wEd73tPVi/08jevxMaTUCY48O+"
    "hF3SQOm0WT4Ey5s/SJ1OPwksPT/6aeU/MAduv6LXIT8HoQs/oPdGv8AA970/sfo9+yufvyA6"
    "LL+wHFo/bCAav3TdEUBio64+ZwZbv9Gsvj9DoAm/IFATv0rlyD9c+4A/NJqKvxJG6D8iS3k+"
    "j+kxQKq+AT++aUy+ovH4P8ANdL+sAiRAhAGzP3xfeL9jjeA/clUWv2oau78Ncoa+i0GwPuqI"
    "lT4wTZI+LxSVP/LkxL/3+w5AcBCLO+Leqz9kxyO/el0Bvyohij+3l0M/felTP3sjWb9Vi8q9"
    "nD0Tv1ExHb4i4do9IPJTP0/viD+kFqS9m8NFvw5YXz/sRdA/E2akvZ6/FL5gUBE/0aqNP7WJ"
    "XkBE5fQ/O7ZkQEybkT+tnFg/IKXcP0L1D7/KoRs/OmArQGfX4D9d/jI/ZKEGPwibsj1NPVW/"
    "mjAYP4F0jT7l1/8+/AAHQKg5Uz6IYZW/5o7Ovd+AGz48pGQ/5Eyfvy54uL6CtHa/b7sLP1Q1"
    "+j8+hwU/hFMnPqcxvj+0KXS/FziWP3gxGEBYi3y/3Ig7QHXeWL8i4vg+8wmvvigCST+3Lw0/"
    "eT2Mv4TafT8bGPE+XfVNQDId/j4Hb+o+Yrf2Px+ZPkCybje/LY6dPjZfBL9ajXE9wDGzvcUS"
    "nb4daiA/q2+Svz5snD2OLEZAHf84QOq7u7/VTXA/MGHjvm+FtD5kp8Y+vPoaQPbReUCZVdU/"
    "pB1EQLWECz8CBoC/Jn43P6hCvL/saEU/kXsZP9S4YECNwnw+L8A5vhyloL97s2JASESSvszs"
    "zL4qypY9ghkNvjAOO0DAwTBABsKmPi2BLL9gzeu+gaIUP+LpyD60uei/VlsuPytFpD8LSvk+"
    "93snv7+tOj/zcom+SpYMv1/0mb6LfZk+cWQOvrcdjj8tR4a+JClmP/7vQ7/eWgJAqDaCP/ok"
    "3T00Zxi/xExqP7Yctb8fnRpANdW0P09MQT6BEJC9DjuWPUDoOb8ZwVE+G5K/v7MDIb6Ml2c/"
    "Np8CQDSSTj+tu3q+vGKxvoHAGj5nG8M7ijr0Ptzy5T/tg+0/BeOHP74z7r8rkCE/3fSWvy9x"
    "9D+eAiW+UBIBP8SnA0DmCWO/fYhKQGf1aD44lhc8Qz46v4jskECU+EY/+cWCvn46Vr8fRQFA"
    "u4C6P4eptL+sndE/F4AxvjcGfL9vBXq9r4Z4PyPxgb68qnY+HKhwv0AvoD/SpQBAb+yUv0TK"
    "ZT8aawc+4rFYvQ6Voz8Zjd0/G8ObPqvkuz/M6hs/1vgPQJBmgz+wAb6+7dTrvm7uiT/E3Fe+"
    "1zijv3/j071EELC/OJjxvJtK4z/c/KhAcuKLv74jl77bPVZAllTRP+AvYb9ae/O+s4KWPx2B"
    "y76QvyQ/cQnFP7Ebrb1jN4W/LUE+v2qTor45NV0/N/pkvxhfRr/egiJAG82rvgn15D+uEZ87"
    "lUbWPgKmfz8OPy++8ACBP1ahUj/Zyog/WFjMPpSKbz/vQ/M+q06fv6KFVj6FysE/cY/zvrIL"
    "lz+nSeo+umgkQBrsKkAp0MU/gNT3Po69Lj+oX6i/ZQ4WP9s9D79Abgi/P6vJPqODX0AwtFQ8"
    "vsVYP2han79l5fK+BsEGP8hwLr9hlCY/WRk5P4njKr+iIxo/WR1Wv3/O4T6wVqM/wGNLQPXv"
    "GjxadoU/v6QYvXSNlz8iPCI/5D11vybMij4aqfo/wEicP9Z5B7/DlNM/O90CvsekMT/Y14Q7"
    "tnIbvx/fiD4WtfU+HcVuPsImdL8lLAg/OjeRPtULPr8ITsY/RgkXv7PKh71FTIS+m9KLv3Sc"
    "Jz8Zy9I/6rE6vvGpYr8Gb4i/Q60gv4H71j75X9S+xjfLPz7u4T5ytiG/XKYDv/qIID6e0MQ+"
    "zVxKP2uCEL/wEHC/sCKVvzojD0AH83K/AFWBP1LlBEA+2T9At6SxQG0SAUD0J4Q/hAcLQMap"
    "sT8cCq0+9LtEQEq+mT/vC0FAk0MUvt47gj+ef3E/PKYevzTm3j8X3C++I7pKP3Cmib/stBe+"
    "6tdkQEg3PD+PqjS884NAP8X4eT9hGL+/70ltv3ywXb7i7mK/p0TkPh4GM0AoyQE/KdAlv80C"
    "lD/45Z4/UHmBv1PiXD/rI38+1Fk7voag474tFkhA1gfYP+Y7Lr5Am2xAwPbIvrHmwD7Wtgw/"
    "dBnXv2UJhT4Pmh2/hCe7vM6jpD5wKQw/uHEfQB6xtr/BNEO+XvSBQNq91L5CSAY/GgmNQO24"
    "GL8sTrW+YulNP4w4Z0AXaPc9I1GCP1vrar+3dEu/9237vgJSiT8L+aY+z1OEP1xRlz+QzeS+"
    "Nj6CP14jyj4GC68+exMZP8obpUAW5wRAzrjwPihPEUBAjzi+kDNCQCNPDj+7kr0/KNgOPw8u"
    "yr7/fv8/SqBKPm88Y0AxurA/cOgAvycR3D85SO6+eEfSviSTAkAtZT4+IprhPFT8Mr4gdzY/"
    "As1YP5/nlr7GxdG8cpWeP2aeWL9uMHJAg4j7O+olQL+U2KY/kFErP/0TRr8AoKS//wEhQAFJ"
    "h0CE+SW/anOVPl0Yjj4f/P8+RC9gP0rK575sJm8/JE8Sv2TmRT4m37++KFXHP4Gf3T8Zt7A/"
    "X35aPmiShz8Uro2/3FEdP+BSFL4/85++I54AQGLdHkAeuvw+R1IqPlWtdr+1T6Y/e8itvg9E"
    "eT6w6X0/Aupev3379j4hPQw+hWTEP33+tz9eoBe/+p2WvwL+gD+MkG4/yWYtPycC3T9fLdc/"
    "CtyNPyR9gr76tHQ/6AtzPziuOz6MEM6+ZEi2PVBLx7/8d0dAcqpnPjLTOEC9wFa/xxfAPjKo"
    "iT+LvwQ/VGu2vn4MCj+5VBc/2ZUUPpDdhz5gGZc/OCREQGl2PD/Ogw0+KgMcPkxtJEASjJQ+"
    "7uzPP/Smmr7DAMW+nD1sPwF3Rj9Ht6w/ZBfFP5g8uj8PdIo/ohcbv6S1ZT/hoUK/YBQYPkiX"
    "9D443cA/aLwWQE5E/j5lHMw+YgVKP1sVX75IvGu+NJlivzw7Qj+spYi+/m9bv8QNy76PfaxA"
    "Upd2P6Z1kr8zgKq+SuelPo7haL/tOWE/Q6gAP0+d1T+XCgZAltGPPyJ+tz8FZr6+uqofP/Vl"
    "or+41oQ/InIXQLnVG76IKTtAm8N9PsQwwj5FOxQ/hzZTPuD25j9csci+pMHXPju7qz4+Ftg+"
    "0j0JQO39yD4uyIO/1AoCv6PTtz+PwF9AwBTFv+qjpr9wbZE/Ou4pQCSokr/O1T8/YnOaP9CC"
    "5j+iP96+zWw/PkQu8L5yGNy+MwvhvfoyRkBTMka/hCKwvigEoz4k9ps+0mNZP/EEjT+Of4Y/"
    "IQZbvmSbiL91vS8+clMFPvdwsj8r4Jo+CH4bP7dPmT8L2VI/KI7WPnKrAr+t2yM/MLonP2eq"
    "s76ixPg9eeOwPn0cg78KEa6+fTEIPm15wD+4Ko4/cOigvSPNFb9vl2++v8fjvJLivz/XfpM9"
    "kivnP8ccQUAck9E/An4Uv9W1QUBcPwK+VooQv/lmgUDvSWM/ynXCP3StcL4hY7C/35ijP3Q2"
    "iz/4tlS/jB6CPwJxmj6WOJ2/T5BMQPQLk7/VQJ0/brF8Phm1Qj/RtQA9L71JP1DCkj9i3SM/"
    "1K48P9SOf7/XyWS/ohEyvwQxur517C+/cM+fP9wS/D8LF1NAVWOBvpuBgL4DsjU/046YP+F2"
    "Ob62/Vk/ZxOJQFgmA79SASZAJbjRvXzlij/4e8A/WByzQBTE6r6WOrC+4lTYPxMzFLuh8xo/"
    "a03Kv1khHr9petM/5/HJP57SeL+yM8+98L4SQAiRuj/bcDC+5rvTvtIDiT8vHEM/TRIgvoSl"
    "QUCx9ZO/w4X3P8JslL1Aldg/qotbQDTS0j7Zsbc+6FeGQAPLg0AdW4a/U+XkPvwkpL86vj1A"
    "EOhRPxiDDb/i0SdADVWFv7NaVz9F48i+k5O3PvIsjj5ETbE/NE2MQE5kgj8AI1C/YOb+viat"
    "Nr+GUy9AsuNJvnRn5j94sVk/f22vPzYfW74FOM89I9mNPhqVID9LDOs/Lk0hQKN9Ib+guN1A"
    "2Tp+v6h+ob9U15S7TTs3PtB7EL6a3QY/ndu0P1zHxj/RFAC+ivHPP8Mlo7/PICA/9VMBwDiL"
    "PUDmpGA/+xl+QGDnF0CaMjK/J04pv3ShlL5UXWM/TMRVv9GTuTzCbCw/6pVfPRgjdb8Dbw5A"
    "BlIxP2AXyT/yfTBAZTYDQNZq0z8J6ms/RxybP6mhQkA+6JS/ObsVPjM7QT+mZnRAPXC5PwRP"
    "ND7CNgRAHqfBP/0/oT41gpQ/ucxFPrezmr/o2bc/1HePP7r16z5XiJK/ifYRQMwdML9Hr2M9"
    "L2UCwNpTMUC98Y0/rn+CPx1+GL4KrTw/snIhQNLFL7/akIy/ENCFPiiEXz2LTAQ+IA2Ev71G"
    "gT7Cvf8+ee6SP3zL874XCsBAdeUbv4Gk1D7YYOQ+4jaUPz6+Fj5m19Y/XlBtPJYXd7/015u+"
    "sI6AQKXGCUCY5sM+sU+lv7BcPD/2ECG/WgHqPzdH/75OekO/0kUYP07uA0DRWSRAEBJDPw9I"
    "Hr/azzu/YwOuP7KYkb/iIz5AoDRtv32yTz4y4eu/ebIHQI6tE74J+RdA+TAhP7tGGkAC1na/"
    "avbkQFgx8b6DRqy+3sB5vxXrnb6RGyo9CYg5v75n9z98+aI/Tx9FPxqFLz/x0No+vhC4P5QA"
    "5T9V0tA/YUUTvnFLAEAejBC/WIk9vh6cdj/sL+0/0nlCP8fSxD+hF8Q+Nq3jPssDkr+NbSy+"
    "iJQVP0ap7T+jcGRAxMF0PoSqpz18njtACt61vifPqz4L9we/7JAPQLvMAT6dXRVAVD9pQL1n"
    "h79p5vw/AJUIQGYw8b3YDP49f4MeQJJ9oD5TvrM+Y7LfvoEJgUBpgC2/G07tPuDmiL85zsO+"
    "OfGdP50mvL38BW4/lo3nvv0Npj6AqJc+PM71P6Wxjr9iGbA/FEa7PnSBUz95PVS+ABEVQNtf"
    "Er4F/Qi/3SY2P1Ddoz3dtcy+6CHnPbQOkL183C6/TcqiP57Dor+sMhM/6mxDP3WuC0C36C8/"
    "xRvcPn4j+771qxu/YIeGPgvoFj7ioXk8kDX1P0nUxj6eZcM+TejeP3MY8T2dy8W/Kz1Qv/Uh"
    "P0DLCIu+WHM1P5p6lb6bJje/oSBlPi8fXL+HdT87/14VPxbZFT9kEg0+ZIkmQL+7HUBLNKE/"
    "pegEPgCLmL12SHw//mcdv+FfLT88T7M+ldjbP286Eb/fNhk/oSGev6jNAb9u586+jU06vzN0"
    "1j/5XilAzKgsvpAvmz+u3rM+Lh9Iv9qhDb5VKR8/TJyNPXyZjD+1yDJANpXCvwzVV0Bq1uW/"
    "nxv6veRxgD9E5ta+tK0MPwjidr8Mn6a/ZW+6P7ZTmj8AXh9AVwk0QBWkSD6skhq/6VfbvrYR"
    "Qz4zYAw/OvzHvlKBVbxi9k87SA1rvzytTb/othQ/LzMlvoU1hUDc7Jm/QNb5Ps756D5uE4A+"
    "aOZaPdQIpUBQk3I/FAiFPpKtG768WCo+o45QPgjq5r6jpq6/O3fHvXLMF0A6MBw/ZwZZvKAs"
    "4r581Jc+GIqpv0+dKkBGrBZA/n02P24dLkCCjL4/CvEXP9BnjUCuKl6/c9d+vxBdTEDi34I+"
    "Mj85v8aKub80Gko+me9Ov/p5xr6+O1xAuK6yP1X0Lb9s8UO/iiTSPoeZC0BSv85AU9a7P8Sd"
    "iD96dXs8CLHnP4rNvL/SLqU/1nluQF6pHD/rFkZASPBsP4Ip0z+T/14+u4IAv/dJGkDU9ydA"
    "UWeLPzQqEj8ivfA/1N5wP/YuOb4S0yNAkrIGQGQDeb3YZVQ+BSOoP3I7qr+M444/CIouQKfw"
    "2D8403xAsXv5P9PTN79r3yA+Wit1P9aeir74ZEJAmpgXPiAXpD/me4Y/vN8hQOaBEz8OrTlA"
    "/TpBQCWx9j+SuNc/wnJyQATmML+eQEm/09GlPwGfAkAsZFy+zICEP5fQir7v4ac/U2ThvqyP"
    "NT//iMs/7lnpvgELBsBQcuQ/hMS4PhNQL0CO3wpAO61Uv2NnUUDbdMw/Hx1lvmgkVj/UjF6+"
    "Pn6kvrgsXT8X12Y/YfKqPp7hEEAXaq8+JJt0vyjjkD+gBEe/FIPyPx5BRT/+e5y+EkbGP4Xj"
    "jD9/hyo/CRjIPyBxD0AOwyO/EYarPtYv6z9BjgA/un/0vQhGf78AsSg/SUiMQEIbQr/7hwQ/"
    "ttYuQPHOIb+T/xBA9j/xPeqVqj5rCEi/NEHHP0UjRr2lYD4/wLEyv5KFYj/agDQ/3zsAQOal"
    "OL59n1tAsvYdP10Iyr3NWmdAHD00QPyyi73u2kdAQKIjP8Pk9D5dRT9AFqM0P5tmTj9QdMW9"
    "UuZOPprOzj/n6Kw/TtkuPSvLBT98GAQ/T5fIPsUYGL5oNSi/CA0PP0+7qjxw9FC/60KRPf0Z"
    "gb6Hum8+X/qMP+bNQz8rcto+yO69P8w4Ij59T2k/pw2+P3vjlr4lgSC+JYzAvLol/j7TWR1A"
    "O2V5PzIciz/mfE4/TnaMPwJxNb4QbFI/+L5QP1rfKL+VUwZAZ4njvvmekr6RdqI/0ZyZPwyN"
    "dr81xUA+8PCqP345KEDhUX5Abq4IQM3M2j/5eARAPuS6v53ooj9txIs/TDdWP06ipj0fJFG+"
    "mg3QvkyZZT/8JWo9m9OrP6b+tT9supm8yYsGPnxvgj+Cpeo/RFh4vuzPJz4OtXo/8B3iPlM5"
    "AUBRbodAd7DTP98Fiz+9Bd8+pgjnPwJn175qRsk/lHl4QEIBiL/k1OU+xz13QJj+gj0k91u+"
    "NNVEP6UbMT7boVK//PoqP9msAj+XGCK/eBMqQF0C9D/OZLS+ajjVvzqsnD5o0BVA3l/GPyzS"
    "lj+HQS09n8CxPxxaoD9ef8o+y2sJv/dvWkCplzE/BboKPihcij7iXJE/IrQTvyMbOb9TlLc/"
    "mvOEv4xhZT/eC3U/V1cNP089BD4Ops493GiJP3agrr63uwS/hqZUP4Lh3T+hGVm/VQ5xP3vW"
    "MD/ABok8xD+Gv6v+P78ahn+/ordRQCitCr8SZ80/EhaQPzfkmUDx6Tm+Uq8CP/+Kzj9TaZy/"
    "bg7HvtzQN7+DVga/PhSUvsdbh76y270+dtqPvkyRjT8Yu0Y/wBDePsal8T+6V2s/eLK/P/Iq"
    "bD+4UAE/b4VxvzYVTUCNG/0+TSWWP/I3MkCAM1e/qpN7vlYPab73qMo/d7onPwLisz8RNTc/"
    "myLMP1Cthz+0Fk4+9mG9P4MNEEAFeCO/UKCLP4jMU0ACtoo+I3q8PunvMr8lDs0+nJidvk7i"
    "b0CKjvM+e8+gP3y5Jz0uEqo90/N7Pw4djr+FTtS+YE2Ov2i7Ez462N8/AWB5PoHQeb73YHA+"
    "JZybv1f76j+fL48/jCpOvm4KSr/WdBlA6rOOvyzkPb1QsYQ/IkqZP/Yt/r54Br4/t5hRQPzb"
    "hL5PHEBA1vZfPuAwzz4npJW+LvnYviRvk78rlT0/Zye7vvpHqLwOn22/wWQ8v/5XDUBERJpA"
    "38lhP4xHrj+P7vc920knQH9eMr8xWr88O5AsP7e5hz+lsTO/GTiyvCNfLD9e3KlAsOeIPhSo"
    "JL8JAbo+ohUiPfqjhT+7HZc/73h9QKXQkD+1f0q/zGQfPlZ5a0C8JfY/Cwk1PpRn5z9cX0k/"
    "XJMHP5z2zz+qII+/QPWevpkizz9r0KhAuBB7vsSNFUD6/0Y/BgxTQLGwEj6mfMA+Q9usP4vn"
    "/j8P1TM/au20vSIjwb9gWoW/n0dnPpkyCkBi3Su/vETZP93APrtEIPM+ChWhPnRZib6j1j4/"
    "/H/0viuPEEDOoe4/BIGLP+bnvECStXq/xyUAP9rW5j+EYRc/hKINvzU5Uj+MZRFATF9Hv3J7"
    "Az7s1Is/bpb2vWVTor4ZbWQ/WumvvyA8/D6UPS4+lFBIv7q+ib7LPGZAoUZEPWGgl77PrQRA"
    "p7DTvnB0nL/Yhz8/nojOv7TqUECfqZA/D0eZvxeiEb9mxX69guvUPh5dB0CmOvQ9dsZMPQ6y"
    "OT8/2Kk/yygyP88fGb+mxuE/mFxev7LPB7+6BSi/QxvXvlEwdDu05sa+eDxYQAyi2r5RkQk/"
    "6NH1vnybpz+wuO4+AgWLP8xzNj9jK2m+xg1+P/mmQEAB+Q++rCOPP6zYkL5mn5Q+ic8fvRiU"
    "9j2v2Tq/8fV9QD2gSD+E8Vi/VCSHv9aQ1kCN6Rc/p0uuPs5Ttr0eWlG/ylPpvnaLST78TO0+"
    "BpwMPlwgYb9Dy4e+/pezP+J6wb/3qL2/d3bEvg86B75M/oo/qMfoPzjXR79/Z2Q/0VEuQCKO"
    "8D+15jG/1iA1PuaGwz9Ap9Q+amSjPwUWxz5UM1A/dPPOPt5GhzyzAF28/kMUv7Dumj7QsYQ+"
    "ejRxPl3RNz9lN6A/fQUKvxWBOD9yO46/6/UvP+LxiT8Smuk/1w8lv8uw+z/MCl8/1u6APsXF"
    "tT8eOV0/JMoFvwvCsD6VhIy9+Z0fPyTyv71wzku+OHlOv4rtyD9PWu4/ZKKXvzDS+D6CUyxA"
    "QVGKP4vUA0CsmiO++SoJQAlwXT8ASd4+a+K3PtQuWD/zNpFAaPqHvwMCur6YCNs/89VmO+2u"
    "JDtatRy/1nGbPhSepD/nph0/MOreP94YLkB3By6+CGXCPaw11r9s9WS/i84UPwd9q711jgS+"
    "2QzoO7pMdr4On/M+8g/GP/iPOr/OxS4/smvDPoD8v739Ae0+ZPBuP7AOxj7s9C6/aDT/P1Kl"
    "sz+IqAu+Bh50vswTxz6b/8s//7cGPxLmqj1LvpI/ruyFPvauLECGvXlA+XQfQDU2ID/24v29"
    "m5kYPiHiqz9Qc9Q+yW2/P8q0REDo7qS/vTDRvopkAT/QQqw/H5CfPgytij7ErTQ/Qjz3P8pL"
    "1z5Zf2E/WM8FP6rMPr8QiIo99GiwPgApuz5b/RlA7S2CP1CwU77cMsW+iZAMQG7WSD+Bq1a/"
    "jnpfPqWwaL5y0PM/VbglP0I3WL5RdjRAcGlPvhQGTr8sZoG/2DYvQDiUB0CaJCw/+L+iPyKK"
    "7D/UIRhAGSKbQEajDr4EwLy9nId4v9aJjT/kv409JTiwv6VVFL9OdEc/SPcBv6xgJ74fXL4/"
    "FRSKv68IbUCRBqw/etH/vphSPEDZj8Y8AVRAvUTN3L6iJ6U+AKBxP0Ho0r41yMI9nb3dPqQw"
    "Oj7ve8K/OZX7voZfj77tf4M+sUecPlwfCr8cuhq/3Lgvv/R/Fr86Vwm/rG0AP+KPDj6u7Zc+"
    "LmSBPtEkjD/qB7K/XLn9vtmhZD+7QKa/wumgvJ20jb5u0ik/8dh4vjOuhUDTMpU/McM7PYBu"
    "BD8GkJg+vRqLvz37zb7lolm+zCKqP9RONEBZkjU/S3Miv/kMqL8LqZw85ejwvcRxzL5iR4g/"
    "dZYXv+SZST9XeUC/DlqZQPKycT49bfc9s3gVQPJpMb97nJc/JHdiv9tvsj+bOOQ/bOAUvxha"
    "br6KXRw/LNVrP+LRg79vEre+nEqPPxj6EUC0uIA/AMPYP6zqJz81+Ek+bRa1vjqy3z+6aGlA"
    "YKNiP9ZJwz/QolE/bWkMP7Oabj4dicm/YrEkv1raVb8fbZU9Fb64PccWab/jTcs9fnXvPsWH"
    "kz+VFGO/LW/5vx8IcT4S1iA/uYKDQA4shD5V8dE+Q1gTP6T6zr0IZZ+/C+KDP0O8Uj4wwpNA"
    "BYBPv29zhT98E7xAozCJv9OKLj8ahWi/OxtyP4pGIEBzl/g/cXTWvkyt3zxEv42/QC0oQLTD"
    "8L52hZQ/bty2vndD5r2Kmly/8v3bvfJIsz/KN5Q+J+4NQASgwj+WvSm+60gOvq3Pvb7b92S/"
    "bcQOQOCk/j+nqVK/jtyav/gzG75dm3s+UFnZvUA1SD814uc9L8IaQCidqr1+r44/JYetvv1/"
    "AkAACCNAThb9P+wBob9M+30/wgC+PmK8zj/jVSo/M8Y3v67Cjz2+kHe/Z5lGv8G9wz1sBx2+"
    "rfg2v/f1tj4jA+I9EIWAvJj+Mj8cpxi/s8SnvK81ob9uPVdAXbWBP4yRyr45uw89Zvq1Pmoh"
    "WT8w4T6+dv0MQKyQ7j7NnRhAgs5Iv3hTJEBeeMU/ueYOP8FYfzymUf2+5JDpPtktWL4udW6/"
    "CED7P69K9L7YVnc/rZ3sPb5Hnz8kaSA/glViP+oph75e9e6+LcPXP2dlaL+fqrS/PG4jvxpY"
    "Wr0IUnk/cvzRPot3qj/CtlC/xKvJvL4rAL9WQhJAUyz/P/QK179E2Qu/Z8WpP/QEDD9BLk8/"
    "/2uJv55P0T9+xdw/D3/YPlB9Lb/yFeI929LXPzkdqD85bAO/GA8pvyFPSr87ek0/u0/SPFl6"
    "Ez2iTMI9ewhuP3ZBOr9ap3W+okaKPwMU5j+IfXY+z9V+v09pyD/3dtK9Hg28P2bXeL8W2si+"
    "J2O2PrJluz/U3C5ADX5TP8Rr7DyyMYI/EE0LPhdc4D52M2Q+cPc7PwiHvD4ip4E/nlQ6P5vP"
    "WT/P0Pg/NGSFv0z+gz+BNg9A/aUcvqpzpT/SQTo/OheJvrErAkBbgpY/flY3v0wh6j5Sso89"
    "si16Pk5dmj3LJjdAosd/P0Y4R79esZu/sDi+Pygptz9UJdI/ncSavjUWub78jXy/bu+kv7EX"
    "GT+fAMU9Vq5uQLUv4L7eBx4/itCAv22C5r9R4KY/v8Kev8CTYr9h/EQ/HJ86QGZg5j8kmUa+"
    "wn2Cvs79P76axiM+BB1jP7tVjj4n6Sc/6uJdvl2MKkBGV04/CjlsQCyPBL9XfF8/jIWXP7HO"
    "CUDKCwG/JTRrPgolbT5mxEi/Sjzkvgllwz488+u+SML0PyaCB76ROhO/LJtvPp5Dtj+kh70+"
    "xFF3P3WLXr8DpUk/IsZwv2ezmj4wlClAjOQDQAW2yz2vLRe/nKw5PwX4sj996kJABon9PwXF"
    "mD8GUfy+znQNPhySCr9C1v2+/hCpvsHM3T9h25k/UXS8P1iUxT8y+mk+IOPbvm/Xhj4DbGC/"
    "Uk4Xv9hzm75XiRFAgyseP8NvRT8mJgW/IwLevktnLj+HAi+/S9Uhv6Xfgr+sw6A+Ara5PlnH"
    "8T/3nmo/RCUSP8niOb+Itk0+k65rv2KCY0BnIQy+tnnwPkMXRr8Hmao/Y6Q1vi6YHEBqFle/"
    "FmjIPs6p0T/2VIi/UPTOPmQpUkAH+l5A2UyJPpgH0z/IDipA0TTYPn/xf76KAPW+2cmHQCWR"
    "rT4EqwO/LZZMP4f70D62Oxw/PVZ7PSlGKz8SUYC/tg9dv5GsKL+x7gdAtaEKvWp4VDz67Rq9"
    "mo50v1PgPb/DVam+CAUWQAJRvz8lmxRAbtTZPzgunD6H8jI9qV+wPTiO9T7USSs/dLn7PmUw"
    "Xj5lnbc/Xt9nvzDoVT8d26G+COTyvkEcjz5IuiY/UNnXvrUY0z8L8CK+2I8Xv5mhlDxDKs8/"
    "+AZ/P6qHtr5YyC+/kKgUPljtnkD3gkc/AasrQAVXGz/rrs4+N0rSP7eBvL2s6qK+ZQ2WvwTP"
    "hj/QwRU/GkGtPpOGrj+unW6/NED3vv8yDr9K6MO/IeiOP4gR877G6TG/RSKhv9hsdT8bfky/"
    "hlOGvN7PFj8kEQI/h6saQOM2H7+mFUNArN4VPr7pbD5YjLs/4uF3vpAzWT9UXso+MWALP0sm"
    "0T5SsEc/c+o6P4N8Mr/wtyZAuZWqv8hYDj+PjIu+GNi/PvfNHEC/EWe/eDlGviXPCz+86ia+"
    "vbW9vk46gUB5M5M+H1MmvpnNekAguGY/Pv8qvyuaZUAsBbQ/qIOsvreKAL1AUWC/A0SzPn3a"
    "3r32v0tAmsNcQHCoBL9Imsc/hxmhvzM96L1o/pW+I9uoPw3sjj5uI8w/8ALTP9eRW79LagNA"
    "vbzJP+aooD4tBIM+RzB1vUN9Jj9GXTs/JCMQQKmYWrrWCOy+8LJ/P7/qi78M+ho+mtyEv0Ar"
    "V7+0x10/cPdTP1sDJb+e1ku/z5upPpYOZT8mAOG/JM27Ph9DCL8ror6+VJB8v5V3Yb/fSa+/"
    "9wkYP+Heqz7bQps/w2ETP3yf/z53Nkq/21pbPgBt7D8b+xE+wq5PP2x8hz/QUwG/vRJHP5La"
    "QD+B80m/D0e7P+UZpD9I9ry+XltXP8fZgr/4FI6/EnODv6G55j2UGCs/fQTSvuaQ4T7Pvo49"
    "6GwCPnDsCEAKDZK/SZ4GPt9LpL/1bgZAnnZEv9UCDD9f/cA/AJl5Pv/cs70GALk+5Vp/vj8k"
    "rj9w0B1AkkFfv1we3LxqHUQ+tGEqP5Zbl7/GpfG9jvwdQJpbJj7hDW0+v9k2vjiOfD/jzGy/"
    "ideEvvboob/FEzS/Do9TPyKyJkDGvIM/A5UBv7C0F79YEks/Q5wNPZKqM74IiZi/ifwwPiBn"
    "xD9L+ZtAAL81vktAEkCBkJk9j0lYv7TjBUDuWkdABKY1PsBdJ79eLTy/zuKTP92amr8n8mm+"
    "obUsvrSdpECvU8m+5G7pPLUDwr/qjxo/dqIgv1DXgb4FJbw+kosnP/S6gj+i7o4/d7L4P76s"
    "3r4b6uE/qgjTvgi2F7+7QJo/7o+5P3ICnT8aXzE+aZ3ZPTzbFb/0QUG/+9rdvZn6QD8+X/o/"
    "5ZRLP10VbUAECF2/F6+XPiJoxL39OkE/TG84vwUuJ79+14q9EE+jP3+ng7/YjaG+QCAdPyJE"
    "zTyNNaw+Y3d9voHd/j2mDdM//fg1QHe7hUBH6cw+BulHPhpp7b5Ux6c/+aZtPqksvT6yaKA+"
    "UvMnP7Toj77UdRBAk3+mP2yqSj402m6/PDkWvu0TIj6LnZY/Dby3PvG1CkCKIwo/Lr54v5R3"
    "0D8qxxu/tp2CvoHxQL9L/ce+TkRTP+0BGr0mP46/YleIQEYmeD/FGGo/9XyQP33Hkz8ZrWE9"
    "AJ3tPq58jL7dU1o/IWjWvgyWVr+6dK5A8qt9v3A+/z92UW0+hdGIP2Uhkj/11nE+SrP3vsGp"
    "fr/ZrmW/EwypvlT+dT7QKwQ/8t1pQAJENEAvKJw+jDcBQE1F3D7n1Je+u3rnvwMYGr5zt/a+"
    "uGYOv3kYvbykK5g//uaAv/d/e0B0Z7g/7iIzP6wVJz8ZxRa/7e2OP/hmO0AaJk6/kydjPVXi"
    "xz66g8k/lXSIPGvlUL9na569U51EQIYo/r1+RkA/b00DPwYyZkD4u9Y/SSOFviKhfkDUiMa+"
    "X1o6vf2ZFkDciaI+FL9ov+88xT6f1ElAflWnvl7l+D6NIfM/asw2P97lf7+rbJs/v3W7P0p6"
    "a76nn1Q/YIZav91oAz56VCm/o1/2PsSCNz/VfZM/aVFHQPyYaT/7hYo/QHHGPu3HE0CsBCs/"
    "bR2+QM++KED2NXc/Bm91PyF3Cj9AM5c/dfqavp0V8L5PDie+W1UnP5qtkD65y0y/83sEv9vK"
    "xD4Q4KQ+OtDFP4utwzzD8LE+hGt4PwDG5b5yqFU9oQluP07Atz/Ah5c//nQov8PDDL3qwWQ9"
    "VW1MQJoCAL/blYQ/yMopv+qLjD+2QUG/ZB4QP8qFhL/Rp6I9mSv/P+sZyT/BeqY/yYPiPjp1"
    "6b4Yw8w+ovUDPoGRO0ByUKU/Ot2UPxqVLECRIWNAkzdavjYmBr66DO0/cc4rv5CbMUCH9aA/"
    "D8IaP4mOgL+la2A/EdzQvmHRWUBtW3i/wMZqP+nTvj8eNg2/rujNP9vgtD8aCSY+gLn1P50A"
    "iT7LNg+/3EL4PqnCZb1FTTq+yJfhPuzp2D4mXrE/9OvUvBEIhr/+dZM/8jukPyk6UL8Atwk/"
    "OGYWvyBTkD7R4Ei/VQkKQMr71T+HM35AuLkcv3WYVb869Qi/zp/JPzneMz91e6I+NC8OQBJe"
    "gj/jGAm/Sr4MP6McrT/JCkq/alKwvnP1wz91rTo/kd+svlK34kCLClA/4EpXv4+gZb96wKpA"
    "5ZjKP/wjsT4+b5u+A7oyPygJNrxLd6o+XoeQvpYBTz3k6sM/QL3pP91Urj7Mcf88/8q5PkyG"
    "Fr7PgBo/sAOIPiFKkD8pjT2+ugISPp3dO74/YxG/ckmhPxFPDj+QcU9A7S2DPmS1xT8DJ9w/"
    "2kD9vuHPmz9hCbw/OU2bvuWeZz9wRgVANMLGP3Mlsz+dtI2+pQyxPqV21z5q0pK/VSHEvtgC"
    "f778bjg/MhnsPhI2G0B22Sk/5QauPsWYr76LRFK+PWqTvi+BD79gdmy/6pPqP2ZVWL8IHew8"
    "1X19v6h8aUDd+P49FGhcv4osQ0DKXQE/iJUfvlTgwT7xc3y/ZytyvjeihEB0//C9fazBP5yO"
    "kb6kgi1AW1/hvVr/9T6ISR4/97ogP/QDcL9wb54+sPSdPwAeI0DRIdE+uno+v7qxjr9CE/0/"
    "4bTqPwe/U7/HMYK9RdtfP3TDFEAJNy+9F36nQEWBY75pd/0/cUhcP545QL5fdw1AWzWHv8ka"
    "l0Bo5Ic+xqcHve8GyD1n/t4/pCnnPPgHNT/RcbY/2pboPh5sgz/B13U/WyqMPoiiab+Bqic9"
    "hFlBP0AzKr5wqDC/XOYxPHUUx76yvARAHYiav9YUTL+km6I+xqLsPnX4rD6Z/3c+Hic3PvQd"
    "ar94cek/oZe1PzZJZb89pq++GNeBvpJEYz+WRaY/cQVCQOSxGD8axu8+EtzCPv12wD969HQ/"
    "vfGJvzyMp76DZ8g/kf47vjFkIL/eZSQ/mkUGP33imj9EKlA+FYnmPaVlbEDphMM/lFEUvja2"
    "Fb4sBQ5Aqtmgv+8hkb9ms7E+3xKmvigJyL6jiBw/N/CoPyg7eT8dpzm/WIxTP4FXPkB5mPq+"
    "lQlQvlbYJr+3BxY/YOxRQEGmGj/DVvO+rrPIPwvrIUCjmJK/kj5hPwLhmb/s4PU+LT8OvaxV"
    "pz5EMj6/dSzYvhqJmr6+5cQ+ZwhbvlCzJz8yZ3G/supoQIH3xz7dDdU+I9WwPprUSb/yYJ6/"
    "4KSkv7ivOkBADaS/xI5jP8vtD79wQnw/dm16P1pTDD+85ug/Zm3HPyIiq782wrw+RWCCP5fK"
    "T79FqIY/204Iv039m7/ncs4/rTTbvpO6PT80/w0/m+y8PwUaBj0kgQC9CQpGP9bVAz+Wq3U/"
    "oa4NQJPCnLzKMUY+vvDsPdU6BUDsUU8/OlJgQJs30z/Itw8+5lI+PPPNNj73vzNACgZiPy8+"
    "db7qTk9A9FUevjRcJb+ivGw/wy6bv/sMrr0="
)


def _constants():
    h = _hamming_ball_np(_BLOCK_SIZE, min(_HAMMING_DIST, _BLOCK_SIZE))
    n_ball = h.shape[0]
    h_pad = np.zeros((_NB, _KP), np.float32)
    h_pad[:n_ball, :_BLOCK_SIZE] = h

    changes_pad = np.zeros((_BATCH, _KP), np.float32)
    changes_pad[:, :_BLOCK_SIZE] = h[_CHOSEN]

    g = np.frombuffer(base64.b64decode("".join(_G_B64)),
                      dtype="<f4").reshape(_BATCH, n_ball)
    g_pad = np.full((_BATCH, _NB), -1e30, np.float32)
    g_pad[:, :n_ball] = g
    return h_pad, changes_pad, g_pad


_H_PAD, _CHANGES_PAD, _G_PAD = _constants()
_CONSTS = np.zeros((256, _KP), np.float32)
_CONSTS[0:64] = _H_PAD
_CONSTS[64:128] = _H_PAD
_CONSTS[64:128, _KP - 1] = 1.0
_CONSTS[128:192] = _CHANGES_PAD
_CONSTS[192:256, :_NB] = _G_PAD
_CONSTS[192:256, _NB:] = -1e30


def _sampler_kernel(x_ref, wt_ref, w2t_ref, c_ref, out_ref):
    x = x_ref[...]
    wt = wt_ref[...]
    w2t = w2t_ref[...]
    h_sel = c_ref[0:_NB, :]
    h_ext = c_ref[_NB:2 * _NB, :]
    changes = c_ref[2 * _NB:3 * _NB, :]
    g = c_ref[3 * _NB:4 * _NB, :_NB]

    xk = x[:, :_KP]
    ub = changes * (1.0 - xk) + (1.0 - changes) * xk
    s = 1.0 - 2.0 * ub

    def _nt(a, b):
        return jax.lax.dot_general(
            a, b, (((1,), (1,)), ((), ())),
            preferred_element_type=jnp.float32)

    base = _nt(x, wt) + _nt(ub - xk, wt[:, :_KP])

    wtk = wt[:, :_KP]
    kidx = jax.lax.broadcasted_iota(jnp.int32, (_BATCH, _HIDDEN, _KP), 2)
    a3 = jnp.where(kidx == _KP - 1, base[:, :, None],
                   s[:, None, :] * wtk[None, :, :])
    act = jnp.maximum(_nt(a3.reshape(_BATCH * _HIDDEN, _KP), h_ext), 0.0)

    ridx = jax.lax.broadcasted_iota(jnp.int32, (_BATCH, _BATCH * _HIDDEN), 0)
    cidx = jax.lax.broadcasted_iota(jnp.int32, (_BATCH, _BATCH * _HIDDEN), 1)
    w2b = jnp.broadcast_to(w2t, (_BATCH, _BATCH * _HIDDEN))
    m_mat = jnp.where(cidx // _HIDDEN == ridx, w2b, 0.0)
    logits = jnp.dot(m_mat, act, preferred_element_type=jnp.float32)

    score = logits + g
    m = jnp.max(score, axis=1, keepdims=True)
    jidx = jax.lax.broadcasted_iota(jnp.int32, (_BATCH, _NB), 1)
    first = jnp.min(jnp.where(score == m, jidx, _NB), axis=1,
                    keepdims=True)
    onehot = (jidx == first).astype(jnp.float32)

    hc = jnp.dot(onehot, h_sel, preferred_element_type=jnp.float32)
    out_block = ub + hc * s
    out_ref[:, :_KP] = out_block
    out_ref[:, _KP:] = x[:, _KP:]


def kernel(x, W1, w2):
    return pl.pallas_call(
        _sampler_kernel,
        out_shape=jax.ShapeDtypeStruct((_BATCH, _DIM), jnp.float32),
    )(x, W1.T, jnp.tile(w2, _BATCH).reshape(1, _BATCH * _HIDDEN), _CONSTS)
